# baseline (speedup 1.0000x reference)
import functools

import jax
import jax.numpy as jnp
from jax import lax
from jax.experimental import pallas as pl
from jax.experimental.pallas import tpu as pltpu
from jax.experimental.pallas import tpu_sc as plsc

N_NODES = 10000
N_EDGES = 320000
G_GRAPHS = 64
EPS = 1e-5

NC, NS, L = 2, 16, 16
NW = NC * NS
EPT = N_EDGES // NW
K_E = 128
NCHUNK = 79
EPT_P = NCHUNK * K_E
E_PAD = NW * EPT_P
WB_TILES = 10
WB_ROWS = N_NODES // WB_TILES
WB_CH = 200


N_PAD = 10240
NPT = N_PAD // NS
EPG = 2 * EPT


def _make_degnorm():
  mesh = plsc.VectorSubcoreMesh(
      core_axis_name="c", subcore_axis_name="s", num_cores=NC, num_subcores=NS)

  @functools.partial(
      pl.kernel,
      out_type=jax.ShapeDtypeStruct((N_EDGES,), jnp.float32),
      mesh=mesh,
      scratch_types=[
          pltpu.VMEM((EPG,), jnp.int32),
          pltpu.VMEM((EPG,), jnp.float32),
          pltpu.VMEM((N_PAD,), jnp.float32),
          pltpu.VMEM((NPT,), jnp.float32),
          pltpu.VMEM((N_PAD,), jnp.float32),
          pltpu.VMEM((EPT,), jnp.int32),
          pltpu.VMEM((EPT,), jnp.int32),
          pltpu.VMEM((EPT,), jnp.float32),
          pltpu.VMEM((EPT,), jnp.float32),
          pltpu.VMEM_SHARED((NS, N_PAD), jnp.float32),
          pltpu.VMEM_SHARED((N_PAD,), jnp.float32),
          pltpu.SemaphoreType.DMA,
      ],
      compiler_params=pltpu.CompilerParams(use_tc_tiling_on_sc=False,
                                           needs_layout_passes=False),
  )
  def degnorm(dst_hbm, src_hbm, w_hbm, out_hbm, dv2, ev2, dlocal, slice_v,
              dinvv, srct, dstt, wt, outv, slab, dinv_sh, msem):
    cid = lax.axis_index("c")
    sid = lax.axis_index("s")
    tid = sid * NC + cid
    zeros = jnp.zeros((L,), jnp.float32)

    pltpu.async_copy(dst_hbm.at[pl.ds(sid * EPG, EPG)], dv2, msem)
    pltpu.async_copy(w_hbm.at[pl.ds(sid * EPG, EPG)], ev2, msem)
    pltpu.async_copy(src_hbm.at[pl.ds(tid * EPT, EPT)], srct, msem)
    pltpu.async_copy(dst_hbm.at[pl.ds(tid * EPT, EPT)], dstt, msem)
    pltpu.async_copy(w_hbm.at[pl.ds(tid * EPT, EPT)], wt, msem)

    def zbody(i, _):
      dlocal[pl.ds(i * L, L)] = zeros
      return 0

    lax.fori_loop(0, N_PAD // L, zbody, 0)
    pltpu.make_async_copy(dst_hbm.at[pl.ds(sid * EPG, EPG)], dv2, msem).wait()
    pltpu.make_async_copy(w_hbm.at[pl.ds(sid * EPG, EPG)], ev2, msem).wait()

    def deg_body(i, _):
      sl = pl.ds(i * L, L)
      plsc.addupdate_scatter(dlocal, [dv2[sl]], ev2[sl])
      return 0

    lax.fori_loop(0, EPG // L, deg_body, 0)
    pltpu.sync_copy(dlocal, slab.at[sid])
    plsc.subcore_barrier()

    base = sid * NPT

    def zslice(i, _):
      slice_v[pl.ds(i * L, L)] = zeros
      return 0

    lax.fori_loop(0, NPT // L, zslice, 0)
    for k in range(NS):
      pltpu.sync_copy(slab.at[k, pl.ds(base, NPT)], dlocal.at[pl.ds(0, NPT)])

      def radd(i, _):
        sl = pl.ds(i * L, L)
        slice_v[sl] = slice_v[sl] + dlocal[sl]
        return 0

      lax.fori_loop(0, NPT // L, radd, 0)

    def rsqrt_body(i, _):
      sl = pl.ds(i * L, L)
      d = slice_v[sl]
      x = jnp.maximum(d, 1e-12)
      bits = plsc.bitcast(x, jnp.int32)
      y = plsc.bitcast(0x5F3759DF - lax.shift_right_logical(bits, 1),
                       jnp.float32)
      for _ in range(3):
        y = y * (1.5 - 0.5 * x * y * y)
      slice_v[sl] = jnp.where(d > 0, y, 0.0)
      return 0

    lax.fori_loop(0, NPT // L, rsqrt_body, 0)
    pltpu.sync_copy(slice_v, dinv_sh.at[pl.ds(base, NPT)])
    plsc.subcore_barrier()
    pltpu.sync_copy(dinv_sh, dinvv)
    pltpu.make_async_copy(src_hbm.at[pl.ds(tid * EPT, EPT)], srct, msem).wait()
    pltpu.make_async_copy(dst_hbm.at[pl.ds(tid * EPT, EPT)], dstt, msem).wait()
    pltpu.make_async_copy(w_hbm.at[pl.ds(tid * EPT, EPT)], wt, msem).wait()

    def norm_body(i, _):
      sl = pl.ds(i * L, L)
      a = plsc.load_gather(dinvv, [dstt[sl]])
      b = plsc.load_gather(dinvv, [srct[sl]])
      outv[sl] = (0.0 - a) * wt[sl] * b
      return 0

    lax.fori_loop(0, EPT // L, norm_body, 0)
    pltpu.sync_copy(outv, out_hbm.at[pl.ds(tid * EPT, EPT)])

  return degnorm


def _make_prop(C):
  mesh = plsc.VectorSubcoreMesh(
      core_axis_name="c", subcore_axis_name="s", num_cores=NC, num_subcores=NS)

  @functools.partial(
      pl.kernel,
      out_type=jax.ShapeDtypeStruct((NC, N_NODES, C), jnp.float32),
      mesh=mesh,
      scratch_types=[
          pltpu.VMEM((NCHUNK, K_E), jnp.int32),
          pltpu.VMEM((NCHUNK, K_E), jnp.int32),
          pltpu.VMEM((NCHUNK, K_E), jnp.float32),
          pltpu.VMEM((2, K_E, C), jnp.float32),
          pltpu.VMEM((WB_CH, C), jnp.float32),
          pltpu.VMEM_SHARED((N_NODES, C), jnp.float32),
          pltpu.SemaphoreType.DMA,
          pltpu.SemaphoreType.DMA,
          pltpu.SemaphoreType.DMA,
          pltpu.SemaphoreType.DMA,
          pltpu.SemaphoreType.DMA,
      ],
      compiler_params=pltpu.CompilerParams(use_tc_tiling_on_sc=False),
  )
  def prop(x_hbm, src_hbm, dst_hbm, w_hbm, out_hbm, srcv, dstv, wv, rows, zb,
           acc, gsem0, gsem1, ssem0, ssem1, msem):
    gsem = (gsem0, gsem1)
    ssem = (ssem0, ssem1)
    cid = lax.axis_index("c")
    sid = lax.axis_index("s")
    tid = sid * NC + cid
    zeros = jnp.zeros((L,), jnp.float32)

    pltpu.async_copy(src_hbm.at[tid], srcv, msem)
    pltpu.async_copy(dst_hbm.at[tid], dstv, msem)
    pltpu.async_copy(w_hbm.at[tid], wv, msem)

    def zrow(r, _):
      for c in range(C // L):
        zb[r, pl.ds(c * L, L)] = zeros
      return 0

    lax.fori_loop(0, WB_CH, zrow, 0)

    @pl.when(sid < WB_TILES)
    def _zero():
      for k in range(WB_ROWS // WB_CH):
        pltpu.sync_copy(zb, acc.at[pl.ds(sid * WB_ROWS + k * WB_CH, WB_CH)])

    pltpu.make_async_copy(src_hbm.at[tid], srcv, msem).wait()
    pltpu.make_async_copy(dst_hbm.at[tid], dstv, msem).wait()
    pltpu.make_async_copy(w_hbm.at[tid], wv, msem).wait()
    plsc.subcore_barrier()

    def scale(i, s):
      def gbody(g, _):
        w16 = wv[i, pl.ds(g * L, L)]
        for j in range(L):
          ws = jnp.full((L,), w16[j], jnp.float32)
          e = g * L + j
          for c in range(C // L):
            sl = pl.ds(c * L, L)
            rows[s, e, sl] = rows[s, e, sl] * ws
        return 0

      lax.fori_loop(0, K_E // L, gbody, 0)

    pltpu.async_copy(x_hbm.at[srcv.at[0]], rows.at[0], gsem[0])

    def outer(o, _):
      for b in range(2):
        i = 2 * o + b
        nb = 1 - b

        @pl.when(i + 1 < NCHUNK)
        def _issue():
          @pl.when(i >= 1)
          def _drain():
            pltpu.make_async_copy(rows.at[nb], acc.at[dstv.at[i - 1]],
                                  ssem[nb]).wait()

          pltpu.async_copy(x_hbm.at[srcv.at[i + 1]], rows.at[nb], gsem[nb])

        pltpu.make_async_copy(x_hbm.at[srcv.at[i]], rows.at[b],
                              gsem[b]).wait()
        scale(i, b)
        pltpu.async_copy(rows.at[b], acc.at[dstv.at[i]], ssem[b], add=True)
      return 0

    lax.fori_loop(0, NCHUNK // 2, outer, 0)
    i_last = NCHUNK - 1
    pltpu.make_async_copy(x_hbm.at[srcv.at[i_last]], rows.at[0],
                          gsem[0]).wait()
    scale(i_last, 0)
    pltpu.async_copy(rows.at[0], acc.at[dstv.at[i_last]], ssem[0], add=True)
    pltpu.make_async_copy(rows.at[1], acc.at[dstv.at[i_last - 1]],
                          ssem[1]).wait()
    pltpu.make_async_copy(rows.at[0], acc.at[dstv.at[i_last]],
                          ssem[0]).wait()
    plsc.subcore_barrier()

    @pl.when(sid < WB_TILES)
    def _writeback():
      for k in range(WB_ROWS // WB_CH):
        off = sid * WB_ROWS + k * WB_CH
        pltpu.sync_copy(acc.at[pl.ds(off, WB_CH)],
                        out_hbm.at[cid, pl.ds(off, WB_CH)])

  return prop


NBLK = 10
BR = N_NODES // NBLK

TPW = 25
NPP = N_NODES // TPW
NEG = -3.0e38


def _add2(p):
  C = p.shape[2]

  def body(p_ref, o_ref):
    o_ref[...] = p_ref[0] + p_ref[1]

  return pl.pallas_call(
      body,
      grid=(NBLK,),
      in_specs=[pl.BlockSpec((2, BR, C), lambda i: (0, i, 0))],
      out_specs=pl.BlockSpec((BR, C), lambda i: (i, 0)),
      out_shape=jax.ShapeDtypeStruct((N_NODES, C), jnp.float32),
  )(p)


def _pre1(x, W1, b1):
  def body(x_ref, w_ref, b_ref, a_ref, bb_ref):
    xb = x_ref[...]
    a_ref[...] = jnp.dot(xb, w_ref[0] - w_ref[2],
                         preferred_element_type=jnp.float32) + b_ref[...]
    bb_ref[...] = jnp.concatenate(
        [jnp.dot(xb, w_ref[1], preferred_element_type=jnp.float32),
         jnp.dot(xb, w_ref[2], preferred_element_type=jnp.float32)], axis=1)

  return pl.pallas_call(
      body,
      grid=(NBLK,),
      in_specs=[
          pl.BlockSpec((BR, 128), lambda i: (i, 0)),
          pl.BlockSpec((3, 128, 16), lambda i: (0, 0, 0)),
          pl.BlockSpec((1, 16), lambda i: (0, 0)),
      ],
      out_specs=[
          pl.BlockSpec((BR, 16), lambda i: (i, 0)),
          pl.BlockSpec((BR, 32), lambda i: (i, 0)),
      ],
      out_shape=[
          jax.ShapeDtypeStruct((N_NODES, 16), jnp.float32),
          jax.ShapeDtypeStruct((N_NODES, 32), jnp.float32),
      ],
  )(x, W1, b1.reshape(1, -1))


def _cheb_u(Xin, Tx1, p2, W, b, with_act=True):
  Cin, Cout = W.shape[1], W.shape[2]

  def body(x_ref, t1_ref, p2_ref, w_ref, b_ref, u_ref, st_ref):
    i = pl.program_id(0)
    xb = x_ref[...]
    t2 = 2.0 * (p2_ref[0] + p2_ref[1]) - xb
    h = (jnp.dot(xb, w_ref[0], preferred_element_type=jnp.float32) +
         jnp.dot(t1_ref[...], w_ref[1], preferred_element_type=jnp.float32) +
         jnp.dot(t2, w_ref[2], preferred_element_type=jnp.float32) +
         b_ref[...])
    if not with_act:
      u_ref[...] = h
      return
    u = jax.nn.softplus(h)
    u_ref[...] = u

    @pl.when(i == 0)
    def _init():
      st_ref[...] = jnp.zeros_like(st_ref)

    st_ref[0:1, :] += jnp.sum(u, axis=0, keepdims=True)
    st_ref[1:2, :] += jnp.sum(u * u, axis=0, keepdims=True)

  return pl.pallas_call(
      body,
      grid=(NBLK,),
      in_specs=[
          pl.BlockSpec((BR, Cin), lambda i: (i, 0)),
          pl.BlockSpec((BR, Cin), lambda i: (i, 0)),
          pl.BlockSpec((2, BR, Cin), lambda i: (0, i, 0)),
          pl.BlockSpec((3, Cin, Cout), lambda i: (0, 0, 0)),
          pl.BlockSpec((1, Cout), lambda i: (0, 0)),
      ],
      out_specs=[
          pl.BlockSpec((BR, Cout), lambda i: (i, 0)),
          pl.BlockSpec((2, Cout), lambda i: (0, 0)),
      ],
      out_shape=[
          jax.ShapeDtypeStruct((N_NODES, Cout), jnp.float32),
          jax.ShapeDtypeStruct((2, Cout), jnp.float32),
      ],
  )(Xin, Tx1, p2, W, b.reshape(1, -1))


def _l1_u(A, U1, pV):
  def body(a_ref, u1_ref, pv_ref, u_ref, st_ref):
    i = pl.program_id(0)
    u = jax.nn.softplus(a_ref[...] + u1_ref[...] +
                        2.0 * (pv_ref[0] + pv_ref[1]))
    u_ref[...] = u

    @pl.when(i == 0)
    def _init():
      st_ref[...] = jnp.zeros_like(st_ref)

    st_ref[0:1, :] += jnp.sum(u, axis=0, keepdims=True)
    st_ref[1:2, :] += jnp.sum(u * u, axis=0, keepdims=True)

  return pl.pallas_call(
      body,
      grid=(NBLK,),
      in_specs=[
          pl.BlockSpec((BR, 16), lambda i: (i, 0)),
          pl.BlockSpec((BR, 16), lambda i: (i, 0)),
          pl.BlockSpec((2, BR, 16), lambda i: (0, i, 0)),
      ],
      out_specs=[
          pl.BlockSpec((BR, 16), lambda i: (i, 0)),
          pl.BlockSpec((2, 16), lambda i: (0, 0)),
      ],
      out_shape=[
          jax.ShapeDtypeStruct((N_NODES, 16), jnp.float32),
          jax.ShapeDtypeStruct((2, 16), jnp.float32),
      ],
  )(A, U1, pV)


def _bn_apply(u, st, gamma, beta):
  C = u.shape[1]

  def body(u_ref, st_ref, g_ref, be_ref, o_ref):
    mu = st_ref[0:1, :] / N_NODES
    var = st_ref[1:2, :] / N_NODES - mu * mu
    s = g_ref[...] * lax.rsqrt(var + EPS)
    o_ref[...] = (u_ref[...] - mu) * s + be_ref[...]

  return pl.pallas_call(
      body,
      grid=(NBLK,),
      in_specs=[
          pl.BlockSpec((BR, C), lambda i: (i, 0)),
          pl.BlockSpec((2, C), lambda i: (0, 0)),
          pl.BlockSpec((1, C), lambda i: (0, 0)),
          pl.BlockSpec((1, C), lambda i: (0, 0)),
      ],
      out_specs=pl.BlockSpec((BR, C), lambda i: (i, 0)),
      out_shape=jax.ShapeDtypeStruct((N_NODES, C), jnp.float32),
  )(u, st, gamma.reshape(1, -1), beta.reshape(1, -1))


def _make_pool():
  mesh = plsc.VectorSubcoreMesh(
      core_axis_name="c", subcore_axis_name="s", num_cores=NC, num_subcores=NS)

  @functools.partial(
      pl.kernel,
      out_type=(jax.ShapeDtypeStruct((TPW, G_GRAPHS, 128), jnp.float32),
                jax.ShapeDtypeStruct((TPW, G_GRAPHS, 128), jnp.float32)),
      mesh=mesh,
      scratch_types=[
          pltpu.VMEM((NPP, 128), jnp.float32),
          pltpu.VMEM((NPP,), jnp.int32),
          pltpu.VMEM((G_GRAPHS, 128), jnp.float32),
          pltpu.VMEM((G_GRAPHS, 128), jnp.float32),
          pltpu.SemaphoreType.DMA,
      ],
      compiler_params=pltpu.CompilerParams(use_tc_tiling_on_sc=False),
  )
  def pool(h_hbm, bi_hbm, omax_hbm, osum_hbm, hbuf, bib, amax, asum, msem):
    cid = lax.axis_index("c")
    sid = lax.axis_index("s")
    tid = sid * NC + cid

    @pl.when(tid < TPW)
    def _work():
      pltpu.async_copy(h_hbm.at[pl.ds(tid * NPP, NPP)], hbuf, msem)
      pltpu.async_copy(bi_hbm.at[pl.ds(tid * NPP, NPP)], bib, msem)
      neg = jnp.full((L,), NEG, jnp.float32)
      zeros = jnp.zeros((L,), jnp.float32)

      def ibody(r, _):
        for c in range(128 // L):
          amax[r, pl.ds(c * L, L)] = neg
          asum[r, pl.ds(c * L, L)] = zeros
        return 0

      lax.fori_loop(0, G_GRAPHS, ibody, 0)
      pltpu.make_async_copy(h_hbm.at[pl.ds(tid * NPP, NPP)], hbuf, msem).wait()
      pltpu.make_async_copy(bi_hbm.at[pl.ds(tid * NPP, NPP)], bib, msem).wait()

      def gbody(g, _):
        b16 = bib[pl.ds(g * L, L)]
        for j in range(L):
          b = b16[j]
          n = g * L + j
          for c in range(128 // L):
            sl = pl.ds(c * L, L)
            v = hbuf[n, sl]
            amax[b, sl] = jnp.maximum(amax[b, sl], v)
            asum[b, sl] = asum[b, sl] + v
        return 0

      lax.fori_loop(0, NPP // L, gbody, 0)
      pltpu.sync_copy(amax, omax_hbm.at[tid])
      pltpu.sync_copy(asum, osum_hbm.at[tid])

  return pool


def _head(pmax, psum, bi, Wd, bd):
  def body(pm_ref, ps_ref, bi_ref, wd_ref, bd_ref, o_ref):
    m = jnp.max(pm_ref[...], axis=0)
    s = jnp.sum(ps_ref[...], axis=0)
    gids = lax.broadcasted_iota(jnp.int32, (G_GRAPHS, N_NODES), 0)
    cnt = jnp.sum((gids == bi_ref[...]).astype(jnp.float32), axis=1)
    cnt = jnp.maximum(cnt, 1.0)
    mean = s / cnt[:, None]
    pooled = jnp.concatenate([m, mean], axis=1)
    logits = jnp.dot(pooled, wd_ref[...],
                     preferred_element_type=jnp.float32) + bd_ref[...]
    mx = jnp.max(logits, axis=-1, keepdims=True)
    lse = mx + jnp.log(jnp.sum(jnp.exp(logits - mx), axis=-1, keepdims=True))
    o_ref[...] = logits - lse

  return pl.pallas_call(
      body,
      out_shape=jax.ShapeDtypeStruct((G_GRAPHS, 4), jnp.float32),
  )(pmax, psum, bi.reshape(1, -1), Wd, bd.reshape(1, -1))


def kernel(x, edge_index, batch_index, edge_weight, W1, b1, W2, b2, W3, b3,
           W4, b4, W5, b5, g1, be1, g2, be2, g3, be3, g4, be4, Wd, bd):
  row, col = edge_index[0], edge_index[1]
  norm_w = _make_degnorm()(row, col, edge_weight)

  npad = E_PAD - N_EDGES
  zi = jnp.zeros((npad,), jnp.int32)
  src_r = jnp.concatenate([col, zi]).reshape(NW, NCHUNK, K_E)
  dst_r = jnp.concatenate([row, zi]).reshape(NW, NCHUNK, K_E)
  w_r = jnp.concatenate([norm_w,
                         jnp.zeros((npad,), jnp.float32)]).reshape(
                             NW, NCHUNK, K_E)

  props = {}

  def Pp(X):
    C = X.shape[1]
    if C not in props:
      props[C] = _make_prop(C)
    return props[C](X, src_r, dst_r, w_r)

  def layer(X, W, b, gamma, beta):
    p1 = Pp(X)
    Tx1 = _add2(p1)
    p2 = Pp(Tx1)
    u, st = _cheb_u(X, Tx1, p2, W, b)
    return _bn_apply(u, st, gamma, beta)

  A, B = _pre1(x, W1, b1)
  U = _add2(Pp(B))
  pV = Pp(U[:, 16:])
  u1, st1 = _l1_u(A, U[:, :16], pV)
  h = _bn_apply(u1, st1, g1, be1)

  h = layer(h, W2, b2, g2, be2)
  h = layer(h, W3, b3, g3, be3)
  h = layer(h, W4, b4, g4, be4)

  Xa, Xb = h[:, :64], h[:, 64:]
  T1a, T1b = _add2(Pp(Xa)), _add2(Pp(Xb))
  Tx1 = jnp.concatenate([T1a, T1b], axis=1)
  p2 = jnp.concatenate([Pp(T1a), Pp(T1b)], axis=2)
  h5, _ = _cheb_u(h, Tx1, p2, W5, b5, with_act=False)

  pmax, psum = _make_pool()(h5, batch_index)
  return _head(pmax, psum, batch_index, Wd, bd)


# speedup vs baseline: 1.1604x; 1.1604x over previous
import functools

import jax
import jax.numpy as jnp
from jax import lax
from jax.experimental import pallas as pl
from jax.experimental.pallas import tpu as pltpu
from jax.experimental.pallas import tpu_sc as plsc

N_NODES = 10000
N_EDGES = 320000
G_GRAPHS = 64
EPS = 1e-5

NC, NS, L = 2, 16, 16
NW = NC * NS
EPT = N_EDGES // NW
K_E = 80
NCHUNK = 125
EPT_P = NCHUNK * K_E
E_PAD = NW * EPT_P
WB_TILES = 10
WB_ROWS = N_NODES // WB_TILES
WB_CH = 200


N_PAD = 10240
NPT = N_PAD // NS
EPG = 2 * EPT


def _make_degnorm():
  mesh = plsc.VectorSubcoreMesh(
      core_axis_name="c", subcore_axis_name="s", num_cores=NC, num_subcores=NS)

  @functools.partial(
      pl.kernel,
      out_type=jax.ShapeDtypeStruct((N_EDGES,), jnp.float32),
      mesh=mesh,
      scratch_types=[
          pltpu.VMEM((EPG,), jnp.int32),
          pltpu.VMEM((EPG,), jnp.float32),
          pltpu.VMEM((N_PAD,), jnp.float32),
          pltpu.VMEM((NPT,), jnp.float32),
          pltpu.VMEM((N_PAD,), jnp.float32),
          pltpu.VMEM((EPT,), jnp.int32),
          pltpu.VMEM((EPT,), jnp.int32),
          pltpu.VMEM((EPT,), jnp.float32),
          pltpu.VMEM((EPT,), jnp.float32),
          pltpu.VMEM_SHARED((NS, N_PAD), jnp.float32),
          pltpu.VMEM_SHARED((N_PAD,), jnp.float32),
          pltpu.SemaphoreType.DMA,
      ],
      compiler_params=pltpu.CompilerParams(use_tc_tiling_on_sc=False,
                                           needs_layout_passes=False),
  )
  def degnorm(dst_hbm, src_hbm, w_hbm, out_hbm, dv2, ev2, dlocal, slice_v,
              dinvv, srct, dstt, wt, outv, slab, dinv_sh, msem):
    cid = lax.axis_index("c")
    sid = lax.axis_index("s")
    tid = sid * NC + cid
    zeros = jnp.zeros((L,), jnp.float32)

    pltpu.async_copy(dst_hbm.at[pl.ds(sid * EPG, EPG)], dv2, msem)
    pltpu.async_copy(w_hbm.at[pl.ds(sid * EPG, EPG)], ev2, msem)
    pltpu.async_copy(src_hbm.at[pl.ds(tid * EPT, EPT)], srct, msem)
    pltpu.async_copy(dst_hbm.at[pl.ds(tid * EPT, EPT)], dstt, msem)
    pltpu.async_copy(w_hbm.at[pl.ds(tid * EPT, EPT)], wt, msem)

    def zbody(i, _):
      dlocal[pl.ds(i * L, L)] = zeros
      return 0

    lax.fori_loop(0, N_PAD // L, zbody, 0)
    pltpu.make_async_copy(dst_hbm.at[pl.ds(sid * EPG, EPG)], dv2, msem).wait()
    pltpu.make_async_copy(w_hbm.at[pl.ds(sid * EPG, EPG)], ev2, msem).wait()

    def deg_body(i, _):
      sl = pl.ds(i * L, L)
      plsc.addupdate_scatter(dlocal, [dv2[sl]], ev2[sl])
      return 0

    lax.fori_loop(0, EPG // L, deg_body, 0)
    pltpu.sync_copy(dlocal, slab.at[sid])
    plsc.subcore_barrier()

    base = sid * NPT

    def zslice(i, _):
      slice_v[pl.ds(i * L, L)] = zeros
      return 0

    lax.fori_loop(0, NPT // L, zslice, 0)
    for k in range(NS):
      pltpu.sync_copy(slab.at[k, pl.ds(base, NPT)], dlocal.at[pl.ds(0, NPT)])

      def radd(i, _):
        sl = pl.ds(i * L, L)
        slice_v[sl] = slice_v[sl] + dlocal[sl]
        return 0

      lax.fori_loop(0, NPT // L, radd, 0)

    def rsqrt_body(i, _):
      sl = pl.ds(i * L, L)
      d = slice_v[sl]
      x = jnp.maximum(d, 1e-12)
      bits = plsc.bitcast(x, jnp.int32)
      y = plsc.bitcast(0x5F3759DF - lax.shift_right_logical(bits, 1),
                       jnp.float32)
      for _ in range(3):
        y = y * (1.5 - 0.5 * x * y * y)
      slice_v[sl] = jnp.where(d > 0, y, 0.0)
      return 0

    lax.fori_loop(0, NPT // L, rsqrt_body, 0)
    pltpu.sync_copy(slice_v, dinv_sh.at[pl.ds(base, NPT)])
    plsc.subcore_barrier()
    pltpu.sync_copy(dinv_sh, dinvv)
    pltpu.make_async_copy(src_hbm.at[pl.ds(tid * EPT, EPT)], srct, msem).wait()
    pltpu.make_async_copy(dst_hbm.at[pl.ds(tid * EPT, EPT)], dstt, msem).wait()
    pltpu.make_async_copy(w_hbm.at[pl.ds(tid * EPT, EPT)], wt, msem).wait()

    def norm_body(i, _):
      sl = pl.ds(i * L, L)
      a = plsc.load_gather(dinvv, [dstt[sl]])
      b = plsc.load_gather(dinvv, [srct[sl]])
      outv[sl] = (0.0 - a) * wt[sl] * b
      return 0

    lax.fori_loop(0, EPT // L, norm_body, 0)
    pltpu.sync_copy(outv, out_hbm.at[pl.ds(tid * EPT, EPT)])

  return degnorm


def _make_prop(C):
  mesh = plsc.VectorSubcoreMesh(
      core_axis_name="c", subcore_axis_name="s", num_cores=NC, num_subcores=NS)

  @functools.partial(
      pl.kernel,
      out_type=jax.ShapeDtypeStruct((NC, N_NODES, C), jnp.float32),
      mesh=mesh,
      scratch_types=[
          pltpu.VMEM((NCHUNK, K_E), jnp.int32),
          pltpu.VMEM((NCHUNK, K_E), jnp.int32),
          pltpu.VMEM((NCHUNK, K_E), jnp.float32),
          pltpu.VMEM((2, K_E, C), jnp.float32),
          pltpu.VMEM((WB_CH, C), jnp.float32),
          pltpu.VMEM_SHARED((N_NODES, C), jnp.float32),
          pltpu.SemaphoreType.DMA,
          pltpu.SemaphoreType.DMA,
          pltpu.SemaphoreType.DMA,
          pltpu.SemaphoreType.DMA,
          pltpu.SemaphoreType.DMA,
      ],
      compiler_params=pltpu.CompilerParams(use_tc_tiling_on_sc=False),
  )
  def prop(x_hbm, src_hbm, dst_hbm, w_hbm, out_hbm, srcv, dstv, wv, rows, zb,
           acc, gsem0, gsem1, ssem0, ssem1, msem):
    gsem = (gsem0, gsem1)
    ssem = (ssem0, ssem1)
    cid = lax.axis_index("c")
    sid = lax.axis_index("s")
    tid = sid * NC + cid
    zeros = jnp.zeros((L,), jnp.float32)

    pltpu.async_copy(src_hbm.at[tid], srcv, msem)
    pltpu.async_copy(dst_hbm.at[tid], dstv, msem)
    pltpu.async_copy(w_hbm.at[tid], wv, msem)

    def zrow(r, _):
      for c in range(C // L):
        zb[r, pl.ds(c * L, L)] = zeros
      return 0

    lax.fori_loop(0, WB_CH, zrow, 0)

    @pl.when(sid < WB_TILES)
    def _zero():
      for k in range(WB_ROWS // WB_CH):
        pltpu.sync_copy(zb, acc.at[pl.ds(sid * WB_ROWS + k * WB_CH, WB_CH)])

    pltpu.make_async_copy(src_hbm.at[tid], srcv, msem).wait()
    pltpu.make_async_copy(dst_hbm.at[tid], dstv, msem).wait()
    pltpu.make_async_copy(w_hbm.at[tid], wv, msem).wait()
    plsc.subcore_barrier()

    def scale(i, s):
      def gbody(g, _):
        w16 = wv[i, pl.ds(g * L, L)]
        for j in range(L):
          ws = jnp.full((L,), w16[j], jnp.float32)
          e = g * L + j
          for c in range(C // L):
            sl = pl.ds(c * L, L)
            rows[s, e, sl] = rows[s, e, sl] * ws
        return 0

      lax.fori_loop(0, K_E // L, gbody, 0)

    pltpu.async_copy(x_hbm.at[srcv.at[0]], rows.at[0], gsem[0])

    def outer(o, _):
      for b in range(2):
        i = 2 * o + b
        nb = 1 - b

        @pl.when(i + 1 < NCHUNK)
        def _issue():
          @pl.when(i >= 1)
          def _drain():
            pltpu.make_async_copy(rows.at[nb], acc.at[dstv.at[i - 1]],
                                  ssem[nb]).wait()

          pltpu.async_copy(x_hbm.at[srcv.at[i + 1]], rows.at[nb], gsem[nb])

        pltpu.make_async_copy(x_hbm.at[srcv.at[i]], rows.at[b],
                              gsem[b]).wait()
        scale(i, b)
        pltpu.async_copy(rows.at[b], acc.at[dstv.at[i]], ssem[b], add=True)
      return 0

    lax.fori_loop(0, NCHUNK // 2, outer, 0)
    i_last = NCHUNK - 1
    pltpu.make_async_copy(x_hbm.at[srcv.at[i_last]], rows.at[0],
                          gsem[0]).wait()
    scale(i_last, 0)
    pltpu.async_copy(rows.at[0], acc.at[dstv.at[i_last]], ssem[0], add=True)
    pltpu.make_async_copy(rows.at[1], acc.at[dstv.at[i_last - 1]],
                          ssem[1]).wait()
    pltpu.make_async_copy(rows.at[0], acc.at[dstv.at[i_last]],
                          ssem[0]).wait()
    plsc.subcore_barrier()

    @pl.when(sid < WB_TILES)
    def _writeback():
      for k in range(WB_ROWS // WB_CH):
        off = sid * WB_ROWS + k * WB_CH
        pltpu.sync_copy(acc.at[pl.ds(off, WB_CH)],
                        out_hbm.at[cid, pl.ds(off, WB_CH)])

  return prop


NBLK = 10
BR = N_NODES // NBLK

TPW = 25
NPP = N_NODES // TPW
NEG = -3.0e38


def _add2(p):
  C = p.shape[2]

  def body(p_ref, o_ref):
    o_ref[...] = p_ref[0] + p_ref[1]

  return pl.pallas_call(
      body,
      grid=(NBLK,),
      in_specs=[pl.BlockSpec((2, BR, C), lambda i: (0, i, 0))],
      out_specs=pl.BlockSpec((BR, C), lambda i: (i, 0)),
      out_shape=jax.ShapeDtypeStruct((N_NODES, C), jnp.float32),
  )(p)


def _pre1(x, W1, b1):
  def body(x_ref, w_ref, b_ref, a_ref, bb_ref):
    xb = x_ref[...]
    a_ref[...] = jnp.dot(xb, w_ref[0] - w_ref[2],
                         preferred_element_type=jnp.float32) + b_ref[...]
    bb_ref[...] = jnp.concatenate(
        [jnp.dot(xb, w_ref[1], preferred_element_type=jnp.float32),
         jnp.dot(xb, w_ref[2], preferred_element_type=jnp.float32)], axis=1)

  return pl.pallas_call(
      body,
      grid=(NBLK,),
      in_specs=[
          pl.BlockSpec((BR, 128), lambda i: (i, 0)),
          pl.BlockSpec((3, 128, 16), lambda i: (0, 0, 0)),
          pl.BlockSpec((1, 16), lambda i: (0, 0)),
      ],
      out_specs=[
          pl.BlockSpec((BR, 16), lambda i: (i, 0)),
          pl.BlockSpec((BR, 32), lambda i: (i, 0)),
      ],
      out_shape=[
          jax.ShapeDtypeStruct((N_NODES, 16), jnp.float32),
          jax.ShapeDtypeStruct((N_NODES, 32), jnp.float32),
      ],
  )(x, W1, b1.reshape(1, -1))


def _cheb_u(Xin, Tx1, p2, W, b, with_act=True):
  Cin, Cout = W.shape[1], W.shape[2]

  def body(x_ref, t1_ref, p2_ref, w_ref, b_ref, u_ref, st_ref):
    i = pl.program_id(0)
    xb = x_ref[...]
    t2 = 2.0 * (p2_ref[0] + p2_ref[1]) - xb
    h = (jnp.dot(xb, w_ref[0], preferred_element_type=jnp.float32) +
         jnp.dot(t1_ref[...], w_ref[1], preferred_element_type=jnp.float32) +
         jnp.dot(t2, w_ref[2], preferred_element_type=jnp.float32) +
         b_ref[...])
    if not with_act:
      u_ref[...] = h
      return
    u = jax.nn.softplus(h)
    u_ref[...] = u

    @pl.when(i == 0)
    def _init():
      st_ref[...] = jnp.zeros_like(st_ref)

    st_ref[0:1, :] += jnp.sum(u, axis=0, keepdims=True)
    st_ref[1:2, :] += jnp.sum(u * u, axis=0, keepdims=True)

  return pl.pallas_call(
      body,
      grid=(NBLK,),
      in_specs=[
          pl.BlockSpec((BR, Cin), lambda i: (i, 0)),
          pl.BlockSpec((BR, Cin), lambda i: (i, 0)),
          pl.BlockSpec((2, BR, Cin), lambda i: (0, i, 0)),
          pl.BlockSpec((3, Cin, Cout), lambda i: (0, 0, 0)),
          pl.BlockSpec((1, Cout), lambda i: (0, 0)),
      ],
      out_specs=[
          pl.BlockSpec((BR, Cout), lambda i: (i, 0)),
          pl.BlockSpec((2, Cout), lambda i: (0, 0)),
      ],
      out_shape=[
          jax.ShapeDtypeStruct((N_NODES, Cout), jnp.float32),
          jax.ShapeDtypeStruct((2, Cout), jnp.float32),
      ],
  )(Xin, Tx1, p2, W, b.reshape(1, -1))


def _l1_u(A, U1, pV):
  def body(a_ref, u1_ref, pv_ref, u_ref, st_ref):
    i = pl.program_id(0)
    u = jax.nn.softplus(a_ref[...] + u1_ref[...] +
                        2.0 * (pv_ref[0] + pv_ref[1]))
    u_ref[...] = u

    @pl.when(i == 0)
    def _init():
      st_ref[...] = jnp.zeros_like(st_ref)

    st_ref[0:1, :] += jnp.sum(u, axis=0, keepdims=True)
    st_ref[1:2, :] += jnp.sum(u * u, axis=0, keepdims=True)

  return pl.pallas_call(
      body,
      grid=(NBLK,),
      in_specs=[
          pl.BlockSpec((BR, 16), lambda i: (i, 0)),
          pl.BlockSpec((BR, 16), lambda i: (i, 0)),
          pl.BlockSpec((2, BR, 16), lambda i: (0, i, 0)),
      ],
      out_specs=[
          pl.BlockSpec((BR, 16), lambda i: (i, 0)),
          pl.BlockSpec((2, 16), lambda i: (0, 0)),
      ],
      out_shape=[
          jax.ShapeDtypeStruct((N_NODES, 16), jnp.float32),
          jax.ShapeDtypeStruct((2, 16), jnp.float32),
      ],
  )(A, U1, pV)


def _bn_apply(u, st, gamma, beta):
  C = u.shape[1]

  def body(u_ref, st_ref, g_ref, be_ref, o_ref):
    mu = st_ref[0:1, :] / N_NODES
    var = st_ref[1:2, :] / N_NODES - mu * mu
    s = g_ref[...] * lax.rsqrt(var + EPS)
    o_ref[...] = (u_ref[...] - mu) * s + be_ref[...]

  return pl.pallas_call(
      body,
      grid=(NBLK,),
      in_specs=[
          pl.BlockSpec((BR, C), lambda i: (i, 0)),
          pl.BlockSpec((2, C), lambda i: (0, 0)),
          pl.BlockSpec((1, C), lambda i: (0, 0)),
          pl.BlockSpec((1, C), lambda i: (0, 0)),
      ],
      out_specs=pl.BlockSpec((BR, C), lambda i: (i, 0)),
      out_shape=jax.ShapeDtypeStruct((N_NODES, C), jnp.float32),
  )(u, st, gamma.reshape(1, -1), beta.reshape(1, -1))


def _make_pool():
  mesh = plsc.VectorSubcoreMesh(
      core_axis_name="c", subcore_axis_name="s", num_cores=NC, num_subcores=NS)

  @functools.partial(
      pl.kernel,
      out_type=(jax.ShapeDtypeStruct((TPW, G_GRAPHS, 128), jnp.float32),
                jax.ShapeDtypeStruct((TPW, G_GRAPHS, 128), jnp.float32)),
      mesh=mesh,
      scratch_types=[
          pltpu.VMEM((NPP, 128), jnp.float32),
          pltpu.VMEM((NPP,), jnp.int32),
          pltpu.VMEM((G_GRAPHS, 128), jnp.float32),
          pltpu.VMEM((G_GRAPHS, 128), jnp.float32),
          pltpu.SemaphoreType.DMA,
      ],
      compiler_params=pltpu.CompilerParams(use_tc_tiling_on_sc=False),
  )
  def pool(h_hbm, bi_hbm, omax_hbm, osum_hbm, hbuf, bib, amax, asum, msem):
    cid = lax.axis_index("c")
    sid = lax.axis_index("s")
    tid = sid * NC + cid

    @pl.when(tid < TPW)
    def _work():
      pltpu.async_copy(h_hbm.at[pl.ds(tid * NPP, NPP)], hbuf, msem)
      pltpu.async_copy(bi_hbm.at[pl.ds(tid * NPP, NPP)], bib, msem)
      neg = jnp.full((L,), NEG, jnp.float32)
      zeros = jnp.zeros((L,), jnp.float32)

      def ibody(r, _):
        for c in range(128 // L):
          amax[r, pl.ds(c * L, L)] = neg
          asum[r, pl.ds(c * L, L)] = zeros
        return 0

      lax.fori_loop(0, G_GRAPHS, ibody, 0)
      pltpu.make_async_copy(h_hbm.at[pl.ds(tid * NPP, NPP)], hbuf, msem).wait()
      pltpu.make_async_copy(bi_hbm.at[pl.ds(tid * NPP, NPP)], bib, msem).wait()

      def gbody(g, _):
        b16 = bib[pl.ds(g * L, L)]
        for j in range(L):
          b = b16[j]
          n = g * L + j
          for c in range(128 // L):
            sl = pl.ds(c * L, L)
            v = hbuf[n, sl]
            amax[b, sl] = jnp.maximum(amax[b, sl], v)
            asum[b, sl] = asum[b, sl] + v
        return 0

      lax.fori_loop(0, NPP // L, gbody, 0)
      pltpu.sync_copy(amax, omax_hbm.at[tid])
      pltpu.sync_copy(asum, osum_hbm.at[tid])

  return pool


def _head(pmax, psum, bi, Wd, bd):
  def body(pm_ref, ps_ref, bi_ref, wd_ref, bd_ref, o_ref):
    m = jnp.max(pm_ref[...], axis=0)
    s = jnp.sum(ps_ref[...], axis=0)
    gids = lax.broadcasted_iota(jnp.int32, (G_GRAPHS, N_NODES), 0)
    cnt = jnp.sum((gids == bi_ref[...]).astype(jnp.float32), axis=1)
    cnt = jnp.maximum(cnt, 1.0)
    mean = s / cnt[:, None]
    pooled = jnp.concatenate([m, mean], axis=1)
    logits = jnp.dot(pooled, wd_ref[...],
                     preferred_element_type=jnp.float32) + bd_ref[...]
    mx = jnp.max(logits, axis=-1, keepdims=True)
    lse = mx + jnp.log(jnp.sum(jnp.exp(logits - mx), axis=-1, keepdims=True))
    o_ref[...] = logits - lse

  return pl.pallas_call(
      body,
      out_shape=jax.ShapeDtypeStruct((G_GRAPHS, 4), jnp.float32),
  )(pmax, psum, bi.reshape(1, -1), Wd, bd.reshape(1, -1))


def kernel(x, edge_index, batch_index, edge_weight, W1, b1, W2, b2, W3, b3,
           W4, b4, W5, b5, g1, be1, g2, be2, g3, be3, g4, be4, Wd, bd):
  row, col = edge_index[0], edge_index[1]
  norm_w = _make_degnorm()(row, col, edge_weight)

  npad = E_PAD - N_EDGES
  zi = jnp.zeros((npad,), jnp.int32)
  src_r = jnp.concatenate([col, zi]).reshape(NW, NCHUNK, K_E)
  dst_r = jnp.concatenate([row, zi]).reshape(NW, NCHUNK, K_E)
  w_r = jnp.concatenate([norm_w,
                         jnp.zeros((npad,), jnp.float32)]).reshape(
                             NW, NCHUNK, K_E)

  props = {}

  def Pp(X):
    C = X.shape[1]
    if C not in props:
      props[C] = _make_prop(C)
    return props[C](X, src_r, dst_r, w_r)

  def layer(X, W, b, gamma, beta):
    p1 = Pp(X)
    Tx1 = _add2(p1)
    p2 = Pp(Tx1)
    u, st = _cheb_u(X, Tx1, p2, W, b)
    return _bn_apply(u, st, gamma, beta)

  A, B = _pre1(x, W1, b1)
  U = _add2(Pp(B))
  pV = Pp(U[:, 16:])
  u1, st1 = _l1_u(A, U[:, :16], pV)
  h = _bn_apply(u1, st1, g1, be1)

  h = layer(h, W2, b2, g2, be2)
  h = layer(h, W3, b3, g3, be3)
  h = layer(h, W4, b4, g4, be4)

  Xa, Xb = h[:, :64], h[:, 64:]
  T1a, T1b = _add2(Pp(Xa)), _add2(Pp(Xb))
  Tx1 = jnp.concatenate([T1a, T1b], axis=1)
  p2 = jnp.concatenate([Pp(T1a), Pp(T1b)], axis=2)
  h5, _ = _cheb_u(h, Tx1, p2, W5, b5, with_act=False)

  pmax, psum = _make_pool()(h5, batch_index)
  return _head(pmax, psum, batch_index, Wd, bd)


# speedup vs baseline: 1.1611x; 1.0006x over previous
import functools

import jax
import jax.numpy as jnp
from jax import lax
from jax.experimental import pallas as pl
from jax.experimental.pallas import tpu as pltpu
from jax.experimental.pallas import tpu_sc as plsc

N_NODES = 10000
N_EDGES = 320000
G_GRAPHS = 64
EPS = 1e-5

NC, NS, L = 2, 16, 16
NW = NC * NS
EPT = N_EDGES // NW
K_E = 80
NCHUNK = 125
EPT_P = NCHUNK * K_E
E_PAD = NW * EPT_P
WB_TILES = 10
WB_ROWS = N_NODES // WB_TILES
WB_CH = 200


N_PAD = 10240
NPT = N_PAD // NS
EPG = 2 * EPT


def _make_degnorm():
  mesh = plsc.VectorSubcoreMesh(
      core_axis_name="c", subcore_axis_name="s", num_cores=NC, num_subcores=NS)

  @functools.partial(
      pl.kernel,
      out_type=jax.ShapeDtypeStruct((N_EDGES,), jnp.float32),
      mesh=mesh,
      scratch_types=[
          pltpu.VMEM((EPG,), jnp.int32),
          pltpu.VMEM((EPG,), jnp.float32),
          pltpu.VMEM((N_PAD,), jnp.float32),
          pltpu.VMEM((NPT,), jnp.float32),
          pltpu.VMEM((N_PAD,), jnp.float32),
          pltpu.VMEM((EPT,), jnp.int32),
          pltpu.VMEM((EPT,), jnp.int32),
          pltpu.VMEM((EPT,), jnp.float32),
          pltpu.VMEM((EPT,), jnp.float32),
          pltpu.VMEM_SHARED((NS, N_PAD), jnp.float32),
          pltpu.VMEM_SHARED((N_PAD,), jnp.float32),
          pltpu.SemaphoreType.DMA,
      ],
      compiler_params=pltpu.CompilerParams(use_tc_tiling_on_sc=False,
                                           needs_layout_passes=False),
  )
  def degnorm(dst_hbm, src_hbm, w_hbm, out_hbm, dv2, ev2, dlocal, slice_v,
              dinvv, srct, dstt, wt, outv, slab, dinv_sh, msem):
    cid = lax.axis_index("c")
    sid = lax.axis_index("s")
    tid = sid * NC + cid
    zeros = jnp.zeros((L,), jnp.float32)

    pltpu.async_copy(dst_hbm.at[pl.ds(sid * EPG, EPG)], dv2, msem)
    pltpu.async_copy(w_hbm.at[pl.ds(sid * EPG, EPG)], ev2, msem)
    pltpu.async_copy(src_hbm.at[pl.ds(tid * EPT, EPT)], srct, msem)
    pltpu.async_copy(dst_hbm.at[pl.ds(tid * EPT, EPT)], dstt, msem)
    pltpu.async_copy(w_hbm.at[pl.ds(tid * EPT, EPT)], wt, msem)

    def zbody(i, _):
      dlocal[pl.ds(i * L, L)] = zeros
      return 0

    lax.fori_loop(0, N_PAD // L, zbody, 0)
    pltpu.make_async_copy(dst_hbm.at[pl.ds(sid * EPG, EPG)], dv2, msem).wait()
    pltpu.make_async_copy(w_hbm.at[pl.ds(sid * EPG, EPG)], ev2, msem).wait()

    def deg_body(i, _):
      sl = pl.ds(i * L, L)
      plsc.addupdate_scatter(dlocal, [dv2[sl]], ev2[sl])
      return 0

    lax.fori_loop(0, EPG // L, deg_body, 0)
    pltpu.sync_copy(dlocal, slab.at[sid])
    plsc.subcore_barrier()

    base = sid * NPT

    def zslice(i, _):
      slice_v[pl.ds(i * L, L)] = zeros
      return 0

    lax.fori_loop(0, NPT // L, zslice, 0)
    for k in range(NS):
      pltpu.sync_copy(slab.at[k, pl.ds(base, NPT)], dlocal.at[pl.ds(0, NPT)])

      def radd(i, _):
        sl = pl.ds(i * L, L)
        slice_v[sl] = slice_v[sl] + dlocal[sl]
        return 0

      lax.fori_loop(0, NPT // L, radd, 0)

    def rsqrt_body(i, _):
      sl = pl.ds(i * L, L)
      d = slice_v[sl]
      x = jnp.maximum(d, 1e-12)
      bits = plsc.bitcast(x, jnp.int32)
      y = plsc.bitcast(0x5F3759DF - lax.shift_right_logical(bits, 1),
                       jnp.float32)
      for _ in range(3):
        y = y * (1.5 - 0.5 * x * y * y)
      slice_v[sl] = jnp.where(d > 0, y, 0.0)
      return 0

    lax.fori_loop(0, NPT // L, rsqrt_body, 0)
    pltpu.sync_copy(slice_v, dinv_sh.at[pl.ds(base, NPT)])
    plsc.subcore_barrier()
    pltpu.sync_copy(dinv_sh, dinvv)
    pltpu.make_async_copy(src_hbm.at[pl.ds(tid * EPT, EPT)], srct, msem).wait()
    pltpu.make_async_copy(dst_hbm.at[pl.ds(tid * EPT, EPT)], dstt, msem).wait()
    pltpu.make_async_copy(w_hbm.at[pl.ds(tid * EPT, EPT)], wt, msem).wait()

    def norm_body(i, _):
      sl = pl.ds(i * L, L)
      a = plsc.load_gather(dinvv, [dstt[sl]])
      b = plsc.load_gather(dinvv, [srct[sl]])
      outv[sl] = (0.0 - a) * wt[sl] * b
      return 0

    lax.fori_loop(0, EPT // L, norm_body, 0)
    pltpu.sync_copy(outv, out_hbm.at[pl.ds(tid * EPT, EPT)])

  return degnorm


def _make_prop(C):
  mesh = plsc.VectorSubcoreMesh(
      core_axis_name="c", subcore_axis_name="s", num_cores=NC, num_subcores=NS)

  @functools.partial(
      pl.kernel,
      out_type=jax.ShapeDtypeStruct((NC, N_NODES, C), jnp.float32),
      mesh=mesh,
      scratch_types=[
          pltpu.VMEM((NCHUNK, K_E), jnp.int32),
          pltpu.VMEM((NCHUNK, K_E), jnp.int32),
          pltpu.VMEM((NCHUNK, K_E), jnp.float32),
          pltpu.VMEM((2, K_E, C), jnp.float32),
          pltpu.VMEM((WB_CH, C), jnp.float32),
          pltpu.VMEM_SHARED((N_NODES, C), jnp.float32),
          pltpu.SemaphoreType.DMA,
          pltpu.SemaphoreType.DMA,
          pltpu.SemaphoreType.DMA,
          pltpu.SemaphoreType.DMA,
          pltpu.SemaphoreType.DMA,
      ],
      compiler_params=pltpu.CompilerParams(use_tc_tiling_on_sc=False),
  )
  def prop(x_hbm, src_hbm, dst_hbm, w_hbm, out_hbm, srcv, dstv, wv, rows, zb,
           acc, gsem0, gsem1, ssem0, ssem1, msem):
    gsem = (gsem0, gsem1)
    ssem = (ssem0, ssem1)
    cid = lax.axis_index("c")
    sid = lax.axis_index("s")
    tid = sid * NC + cid
    zeros = jnp.zeros((L,), jnp.float32)

    pltpu.async_copy(src_hbm.at[tid], srcv, msem)
    pltpu.async_copy(dst_hbm.at[tid], dstv, msem)
    pltpu.async_copy(w_hbm.at[tid], wv, msem)

    def zrow(r, _):
      for c in range(C // L):
        zb[r, pl.ds(c * L, L)] = zeros
      return 0

    lax.fori_loop(0, WB_CH, zrow, 0)

    @pl.when(sid < WB_TILES)
    def _zero():
      for k in range(WB_ROWS // WB_CH):
        pltpu.sync_copy(zb, acc.at[pl.ds(sid * WB_ROWS + k * WB_CH, WB_CH)])

    pltpu.make_async_copy(src_hbm.at[tid], srcv, msem).wait()
    pltpu.make_async_copy(dst_hbm.at[tid], dstv, msem).wait()
    pltpu.make_async_copy(w_hbm.at[tid], wv, msem).wait()
    plsc.subcore_barrier()

    def scale(i, s):
      def gbody(g, _):
        w16 = wv[i, pl.ds(g * L, L)]
        for j in range(L):
          ws = jnp.full((L,), w16[j], jnp.float32)
          e = g * L + j
          for c in range(C // L):
            sl = pl.ds(c * L, L)
            rows[s, e, sl] = rows[s, e, sl] * ws
        return 0

      lax.fori_loop(0, K_E // L, gbody, 0)

    pltpu.async_copy(x_hbm.at[srcv.at[0]], rows.at[0], gsem[0])

    def outer(o, _):
      for b in range(2):
        i = 2 * o + b
        nb = 1 - b

        @pl.when(i + 1 < NCHUNK)
        def _issue():
          @pl.when(i >= 1)
          def _drain():
            pltpu.make_async_copy(rows.at[nb], acc.at[dstv.at[i - 1]],
                                  ssem[nb]).wait()

          pltpu.async_copy(x_hbm.at[srcv.at[i + 1]], rows.at[nb], gsem[nb])

        pltpu.make_async_copy(x_hbm.at[srcv.at[i]], rows.at[b],
                              gsem[b]).wait()
        scale(i, b)
        pltpu.async_copy(rows.at[b], acc.at[dstv.at[i]], ssem[b], add=True)
      return 0

    lax.fori_loop(0, NCHUNK // 2, outer, 0)
    i_last = NCHUNK - 1
    pltpu.make_async_copy(x_hbm.at[srcv.at[i_last]], rows.at[0],
                          gsem[0]).wait()
    scale(i_last, 0)
    pltpu.async_copy(rows.at[0], acc.at[dstv.at[i_last]], ssem[0], add=True)
    pltpu.make_async_copy(rows.at[1], acc.at[dstv.at[i_last - 1]],
                          ssem[1]).wait()
    pltpu.make_async_copy(rows.at[0], acc.at[dstv.at[i_last]],
                          ssem[0]).wait()
    plsc.subcore_barrier()

    @pl.when(sid < WB_TILES)
    def _writeback():
      for k in range(WB_ROWS // WB_CH):
        off = sid * WB_ROWS + k * WB_CH
        pltpu.sync_copy(acc.at[pl.ds(off, WB_CH)],
                        out_hbm.at[cid, pl.ds(off, WB_CH)])

  return prop


NBLK = 10
BR = N_NODES // NBLK

TPW = 25
NPP = N_NODES // TPW
NEG = -3.0e38


def _add2(p):
  C = p.shape[2]

  def body(p_ref, o_ref):
    o_ref[...] = p_ref[0] + p_ref[1]

  return pl.pallas_call(
      body,
      grid=(NBLK,),
      in_specs=[pl.BlockSpec((2, BR, C), lambda i: (0, i, 0))],
      out_specs=pl.BlockSpec((BR, C), lambda i: (i, 0)),
      out_shape=jax.ShapeDtypeStruct((N_NODES, C), jnp.float32),
  )(p)


def _pre1(x, W1, b1):
  def body(x_ref, w_ref, b_ref, a_ref, bb_ref):
    xb = x_ref[...]
    a_ref[...] = jnp.dot(xb, w_ref[0] - w_ref[2],
                         preferred_element_type=jnp.float32) + b_ref[...]
    bb_ref[...] = jnp.concatenate(
        [jnp.dot(xb, w_ref[1], preferred_element_type=jnp.float32),
         jnp.dot(xb, w_ref[2], preferred_element_type=jnp.float32)], axis=1)

  return pl.pallas_call(
      body,
      grid=(NBLK,),
      in_specs=[
          pl.BlockSpec((BR, 128), lambda i: (i, 0)),
          pl.BlockSpec((3, 128, 16), lambda i: (0, 0, 0)),
          pl.BlockSpec((1, 16), lambda i: (0, 0)),
      ],
      out_specs=[
          pl.BlockSpec((BR, 16), lambda i: (i, 0)),
          pl.BlockSpec((BR, 32), lambda i: (i, 0)),
      ],
      out_shape=[
          jax.ShapeDtypeStruct((N_NODES, 16), jnp.float32),
          jax.ShapeDtypeStruct((N_NODES, 32), jnp.float32),
      ],
  )(x, W1, b1.reshape(1, -1))


def _cheb_u(Xin, Tx1, p2, W, b, with_act=True):
  Cin, Cout = W.shape[1], W.shape[2]

  def body(x_ref, t1_ref, p2_ref, w_ref, b_ref, u_ref, st_ref):
    i = pl.program_id(0)
    xb = x_ref[...]
    t2 = 2.0 * (p2_ref[0] + p2_ref[1]) - xb
    h = (jnp.dot(xb, w_ref[0], preferred_element_type=jnp.float32) +
         jnp.dot(t1_ref[...], w_ref[1], preferred_element_type=jnp.float32) +
         jnp.dot(t2, w_ref[2], preferred_element_type=jnp.float32) +
         b_ref[...])
    if not with_act:
      u_ref[...] = h
      return
    u = jax.nn.softplus(h)
    u_ref[...] = u

    @pl.when(i == 0)
    def _init():
      st_ref[...] = jnp.zeros_like(st_ref)

    st_ref[0:1, :] += jnp.sum(u, axis=0, keepdims=True)
    st_ref[1:2, :] += jnp.sum(u * u, axis=0, keepdims=True)

  return pl.pallas_call(
      body,
      grid=(NBLK,),
      in_specs=[
          pl.BlockSpec((BR, Cin), lambda i: (i, 0)),
          pl.BlockSpec((BR, Cin), lambda i: (i, 0)),
          pl.BlockSpec((2, BR, Cin), lambda i: (0, i, 0)),
          pl.BlockSpec((3, Cin, Cout), lambda i: (0, 0, 0)),
          pl.BlockSpec((1, Cout), lambda i: (0, 0)),
      ],
      out_specs=[
          pl.BlockSpec((BR, Cout), lambda i: (i, 0)),
          pl.BlockSpec((2, Cout), lambda i: (0, 0)),
      ],
      out_shape=[
          jax.ShapeDtypeStruct((N_NODES, Cout), jnp.float32),
          jax.ShapeDtypeStruct((2, Cout), jnp.float32),
      ],
  )(Xin, Tx1, p2, W, b.reshape(1, -1))


def _l1_u(A, U1, pV):
  def body(a_ref, u1_ref, pv_ref, u_ref, st_ref):
    i = pl.program_id(0)
    u = jax.nn.softplus(a_ref[...] + u1_ref[...] +
                        2.0 * (pv_ref[0] + pv_ref[1]))
    u_ref[...] = u

    @pl.when(i == 0)
    def _init():
      st_ref[...] = jnp.zeros_like(st_ref)

    st_ref[0:1, :] += jnp.sum(u, axis=0, keepdims=True)
    st_ref[1:2, :] += jnp.sum(u * u, axis=0, keepdims=True)

  return pl.pallas_call(
      body,
      grid=(NBLK,),
      in_specs=[
          pl.BlockSpec((BR, 16), lambda i: (i, 0)),
          pl.BlockSpec((BR, 16), lambda i: (i, 0)),
          pl.BlockSpec((2, BR, 16), lambda i: (0, i, 0)),
      ],
      out_specs=[
          pl.BlockSpec((BR, 16), lambda i: (i, 0)),
          pl.BlockSpec((2, 16), lambda i: (0, 0)),
      ],
      out_shape=[
          jax.ShapeDtypeStruct((N_NODES, 16), jnp.float32),
          jax.ShapeDtypeStruct((2, 16), jnp.float32),
      ],
  )(A, U1, pV)


def _bn_apply(u, st, gamma, beta):
  C = u.shape[1]

  def body(u_ref, st_ref, g_ref, be_ref, o_ref):
    mu = st_ref[0:1, :] / N_NODES
    var = st_ref[1:2, :] / N_NODES - mu * mu
    s = g_ref[...] * lax.rsqrt(var + EPS)
    o_ref[...] = (u_ref[...] - mu) * s + be_ref[...]

  return pl.pallas_call(
      body,
      grid=(NBLK,),
      in_specs=[
          pl.BlockSpec((BR, C), lambda i: (i, 0)),
          pl.BlockSpec((2, C), lambda i: (0, 0)),
          pl.BlockSpec((1, C), lambda i: (0, 0)),
          pl.BlockSpec((1, C), lambda i: (0, 0)),
      ],
      out_specs=pl.BlockSpec((BR, C), lambda i: (i, 0)),
      out_shape=jax.ShapeDtypeStruct((N_NODES, C), jnp.float32),
  )(u, st, gamma.reshape(1, -1), beta.reshape(1, -1))


def _make_pool():
  mesh = plsc.VectorSubcoreMesh(
      core_axis_name="c", subcore_axis_name="s", num_cores=NC, num_subcores=NS)

  @functools.partial(
      pl.kernel,
      out_type=(jax.ShapeDtypeStruct((TPW, G_GRAPHS, 128), jnp.float32),
                jax.ShapeDtypeStruct((TPW, G_GRAPHS, 128), jnp.float32)),
      mesh=mesh,
      scratch_types=[
          pltpu.VMEM((NPP, 128), jnp.float32),
          pltpu.VMEM((NPP,), jnp.int32),
          pltpu.VMEM((G_GRAPHS, 128), jnp.float32),
          pltpu.VMEM((G_GRAPHS, 128), jnp.float32),
          pltpu.SemaphoreType.DMA,
      ],
      compiler_params=pltpu.CompilerParams(use_tc_tiling_on_sc=False),
  )
  def pool(h_hbm, bi_hbm, omax_hbm, osum_hbm, hbuf, bib, amax, asum, msem):
    cid = lax.axis_index("c")
    sid = lax.axis_index("s")
    tid = sid * NC + cid

    @pl.when(tid < TPW)
    def _work():
      pltpu.async_copy(h_hbm.at[pl.ds(tid * NPP, NPP)], hbuf, msem)
      pltpu.async_copy(bi_hbm.at[pl.ds(tid * NPP, NPP)], bib, msem)
      neg = jnp.full((L,), NEG, jnp.float32)
      zeros = jnp.zeros((L,), jnp.float32)

      def ibody(r, _):
        for c in range(128 // L):
          amax[r, pl.ds(c * L, L)] = neg
          asum[r, pl.ds(c * L, L)] = zeros
        return 0

      lax.fori_loop(0, G_GRAPHS, ibody, 0)
      pltpu.make_async_copy(h_hbm.at[pl.ds(tid * NPP, NPP)], hbuf, msem).wait()
      pltpu.make_async_copy(bi_hbm.at[pl.ds(tid * NPP, NPP)], bib, msem).wait()

      def gbody(g, _):
        b16 = bib[pl.ds(g * L, L)]
        for j in range(L):
          b = b16[j]
          n = g * L + j
          for c in range(128 // L):
            sl = pl.ds(c * L, L)
            v = hbuf[n, sl]
            amax[b, sl] = jnp.maximum(amax[b, sl], v)
            asum[b, sl] = asum[b, sl] + v
        return 0

      lax.fori_loop(0, NPP // L, gbody, 0)
      pltpu.sync_copy(amax, omax_hbm.at[tid])
      pltpu.sync_copy(asum, osum_hbm.at[tid])

  return pool


def _head(pmax, psum, bi, Wd, bd):
  def body(pm_ref, ps_ref, bi_ref, wd_ref, bd_ref, o_ref):
    m = jnp.max(pm_ref[...], axis=0)
    s = jnp.sum(ps_ref[...], axis=0)
    gids = lax.broadcasted_iota(jnp.int32, (G_GRAPHS, N_NODES), 0)
    cnt = jnp.sum((gids == bi_ref[...]).astype(jnp.float32), axis=1)
    cnt = jnp.maximum(cnt, 1.0)
    mean = s / cnt[:, None]
    pooled = jnp.concatenate([m, mean], axis=1)
    logits = jnp.dot(pooled, wd_ref[...],
                     preferred_element_type=jnp.float32) + bd_ref[...]
    mx = jnp.max(logits, axis=-1, keepdims=True)
    lse = mx + jnp.log(jnp.sum(jnp.exp(logits - mx), axis=-1, keepdims=True))
    o_ref[...] = logits - lse

  return pl.pallas_call(
      body,
      out_shape=jax.ShapeDtypeStruct((G_GRAPHS, 4), jnp.float32),
  )(pmax, psum, bi.reshape(1, -1), Wd, bd.reshape(1, -1))


def kernel(x, edge_index, batch_index, edge_weight, W1, b1, W2, b2, W3, b3,
           W4, b4, W5, b5, g1, be1, g2, be2, g3, be3, g4, be4, Wd, bd):
  row, col = edge_index[0], edge_index[1]
  norm_w = _make_degnorm()(row, col, edge_weight)

  src_r = col.reshape(NW, NCHUNK, K_E)
  dst_r = row.reshape(NW, NCHUNK, K_E)
  w_r = norm_w.reshape(NW, NCHUNK, K_E)

  props = {}

  def Pp(X):
    C = X.shape[1]
    if C not in props:
      props[C] = _make_prop(C)
    return props[C](X, src_r, dst_r, w_r)

  def layer(X, W, b, gamma, beta):
    p1 = Pp(X)
    Tx1 = _add2(p1)
    p2 = Pp(Tx1)
    u, st = _cheb_u(X, Tx1, p2, W, b)
    return _bn_apply(u, st, gamma, beta)

  A, B = _pre1(x, W1, b1)
  U = _add2(Pp(B))
  pV = Pp(U[:, 16:])
  u1, st1 = _l1_u(A, U[:, :16], pV)
  h = _bn_apply(u1, st1, g1, be1)

  h = layer(h, W2, b2, g2, be2)
  h = layer(h, W3, b3, g3, be3)
  h = layer(h, W4, b4, g4, be4)

  Xa, Xb = h[:, :64], h[:, 64:]
  T1a, T1b = _add2(Pp(Xa)), _add2(Pp(Xb))
  Tx1 = jnp.concatenate([T1a, T1b], axis=1)
  p2 = jnp.concatenate([Pp(T1a), Pp(T1b)], axis=2)
  h5, _ = _cheb_u(h, Tx1, p2, W5, b5, with_act=False)

  pmax, psum = _make_pool()(h5, batch_index)
  return _head(pmax, psum, batch_index, Wd, bd)


# speedup vs baseline: 1.5460x; 1.3315x over previous
import functools

import jax
import jax.numpy as jnp
from jax import lax
from jax.experimental import pallas as pl
from jax.experimental.pallas import tpu as pltpu
from jax.experimental.pallas import tpu_sc as plsc

N_NODES = 10000
N_EDGES = 320000
G_GRAPHS = 64
EPS = 1e-5

NC, NS, L = 2, 16, 16
NW = NC * NS
EPT = N_EDGES // NW
K_E = 80
NCHUNK = 125
RING = 5
LEAD = 3
WB_TILES = 10
WB_ROWS = N_NODES // WB_TILES
WB_CH = 200


N_PAD = 10240
NPT = N_PAD // NS
EPG = 2 * EPT


def _make_degnorm():
  mesh = plsc.VectorSubcoreMesh(
      core_axis_name="c", subcore_axis_name="s", num_cores=NC, num_subcores=NS)

  @functools.partial(
      pl.kernel,
      out_type=jax.ShapeDtypeStruct((N_EDGES,), jnp.float32),
      mesh=mesh,
      scratch_types=[
          pltpu.VMEM((EPG,), jnp.int32),
          pltpu.VMEM((EPG,), jnp.float32),
          pltpu.VMEM((N_PAD,), jnp.float32),
          pltpu.VMEM((NPT,), jnp.float32),
          pltpu.VMEM((N_PAD,), jnp.float32),
          pltpu.VMEM((EPT,), jnp.int32),
          pltpu.VMEM((EPT,), jnp.int32),
          pltpu.VMEM((EPT,), jnp.float32),
          pltpu.VMEM((EPT,), jnp.float32),
          pltpu.VMEM_SHARED((NS, N_PAD), jnp.float32),
          pltpu.VMEM_SHARED((N_PAD,), jnp.float32),
          pltpu.SemaphoreType.DMA,
      ],
      compiler_params=pltpu.CompilerParams(use_tc_tiling_on_sc=False,
                                           needs_layout_passes=False),
  )
  def degnorm(dst_hbm, src_hbm, w_hbm, out_hbm, dv2, ev2, dlocal, slice_v,
              dinvv, srct, dstt, wt, outv, slab, dinv_sh, msem):
    cid = lax.axis_index("c")
    sid = lax.axis_index("s")
    tid = sid * NC + cid
    zeros = jnp.zeros((L,), jnp.float32)

    pltpu.async_copy(dst_hbm.at[pl.ds(sid * EPG, EPG)], dv2, msem)
    pltpu.async_copy(w_hbm.at[pl.ds(sid * EPG, EPG)], ev2, msem)
    pltpu.async_copy(src_hbm.at[pl.ds(tid * EPT, EPT)], srct, msem)
    pltpu.async_copy(dst_hbm.at[pl.ds(tid * EPT, EPT)], dstt, msem)
    pltpu.async_copy(w_hbm.at[pl.ds(tid * EPT, EPT)], wt, msem)

    def zbody(i, _):
      dlocal[pl.ds(i * L, L)] = zeros
      return 0

    lax.fori_loop(0, N_PAD // L, zbody, 0)
    pltpu.make_async_copy(dst_hbm.at[pl.ds(sid * EPG, EPG)], dv2, msem).wait()
    pltpu.make_async_copy(w_hbm.at[pl.ds(sid * EPG, EPG)], ev2, msem).wait()

    def deg_body(i, _):
      sl = pl.ds(i * L, L)
      plsc.addupdate_scatter(dlocal, [dv2[sl]], ev2[sl])
      return 0

    lax.fori_loop(0, EPG // L, deg_body, 0)
    pltpu.sync_copy(dlocal, slab.at[sid])
    plsc.subcore_barrier()

    base = sid * NPT

    def zslice(i, _):
      slice_v[pl.ds(i * L, L)] = zeros
      return 0

    lax.fori_loop(0, NPT // L, zslice, 0)
    for k in range(NS):
      pltpu.sync_copy(slab.at[k, pl.ds(base, NPT)], dlocal.at[pl.ds(0, NPT)])

      def radd(i, _):
        sl = pl.ds(i * L, L)
        slice_v[sl] = slice_v[sl] + dlocal[sl]
        return 0

      lax.fori_loop(0, NPT // L, radd, 0)

    def rsqrt_body(i, _):
      sl = pl.ds(i * L, L)
      d = slice_v[sl]
      x = jnp.maximum(d, 1e-12)
      bits = plsc.bitcast(x, jnp.int32)
      y = plsc.bitcast(0x5F3759DF - lax.shift_right_logical(bits, 1),
                       jnp.float32)
      for _ in range(3):
        y = y * (1.5 - 0.5 * x * y * y)
      slice_v[sl] = jnp.where(d > 0, y, 0.0)
      return 0

    lax.fori_loop(0, NPT // L, rsqrt_body, 0)
    pltpu.sync_copy(slice_v, dinv_sh.at[pl.ds(base, NPT)])
    plsc.subcore_barrier()
    pltpu.sync_copy(dinv_sh, dinvv)
    pltpu.make_async_copy(src_hbm.at[pl.ds(tid * EPT, EPT)], srct, msem).wait()
    pltpu.make_async_copy(dst_hbm.at[pl.ds(tid * EPT, EPT)], dstt, msem).wait()
    pltpu.make_async_copy(w_hbm.at[pl.ds(tid * EPT, EPT)], wt, msem).wait()

    def norm_body(i, _):
      sl = pl.ds(i * L, L)
      a = plsc.load_gather(dinvv, [dstt[sl]])
      b = plsc.load_gather(dinvv, [srct[sl]])
      outv[sl] = (0.0 - a) * wt[sl] * b
      return 0

    lax.fori_loop(0, EPT // L, norm_body, 0)
    pltpu.sync_copy(outv, out_hbm.at[pl.ds(tid * EPT, EPT)])

  return degnorm


def _make_prop(C):
  mesh = plsc.VectorSubcoreMesh(
      core_axis_name="c", subcore_axis_name="s", num_cores=NC, num_subcores=NS)

  @functools.partial(
      pl.kernel,
      out_type=jax.ShapeDtypeStruct((NC, N_NODES, C), jnp.float32),
      mesh=mesh,
      scratch_types=[
          pltpu.VMEM((NCHUNK, K_E), jnp.int32),
          pltpu.VMEM((NCHUNK, K_E), jnp.int32),
          pltpu.VMEM((NCHUNK, K_E), jnp.float32),
          pltpu.VMEM((RING, K_E, C), jnp.float32),
          pltpu.VMEM((WB_CH, C), jnp.float32),
          pltpu.VMEM_SHARED((N_NODES, C), jnp.float32),
      ] + [pltpu.SemaphoreType.DMA] * (2 * RING) + [
          pltpu.SemaphoreType.DMA,
      ],
      compiler_params=pltpu.CompilerParams(use_tc_tiling_on_sc=False),
  )
  def prop(x_hbm, src_hbm, dst_hbm, w_hbm, out_hbm, srcv, dstv, wv, rows, zb,
           acc, *sems):
    gsem = sems[:RING]
    ssem = sems[RING:2 * RING]
    msem = sems[2 * RING]
    cid = lax.axis_index("c")
    sid = lax.axis_index("s")
    tid = sid * NC + cid
    zeros = jnp.zeros((L,), jnp.float32)

    pltpu.async_copy(src_hbm.at[tid], srcv, msem)
    pltpu.async_copy(dst_hbm.at[tid], dstv, msem)
    pltpu.async_copy(w_hbm.at[tid], wv, msem)

    def zrow(r, _):
      for c in range(C // L):
        zb[r, pl.ds(c * L, L)] = zeros
      return 0

    lax.fori_loop(0, WB_CH, zrow, 0)

    @pl.when(sid < WB_TILES)
    def _zero():
      for k in range(WB_ROWS // WB_CH):
        pltpu.sync_copy(zb, acc.at[pl.ds(sid * WB_ROWS + k * WB_CH, WB_CH)])

    pltpu.make_async_copy(src_hbm.at[tid], srcv, msem).wait()
    pltpu.make_async_copy(dst_hbm.at[tid], dstv, msem).wait()
    pltpu.make_async_copy(w_hbm.at[tid], wv, msem).wait()
    plsc.subcore_barrier()

    def scale(i, s):
      def gbody(g, _):
        w16 = wv[i, pl.ds(g * L, L)]
        for j in range(L):
          ws = jnp.full((L,), w16[j], jnp.float32)
          e = g * L + j
          for c in range(C // L):
            sl = pl.ds(c * L, L)
            rows[s, e, sl] = rows[s, e, sl] * ws
        return 0

      lax.fori_loop(0, K_E // L, gbody, 0)

    for b in range(LEAD):
      pltpu.async_copy(x_hbm.at[srcv.at[b]], rows.at[b], gsem[b])

    def outer(o, _):
      for b in range(RING):
        i = RING * o + b
        js = (b + LEAD) % RING

        @pl.when(i + LEAD < NCHUNK)
        def _issue():
          @pl.when(i >= RING - LEAD)
          def _drain():
            pltpu.make_async_copy(rows.at[js],
                                  acc.at[dstv.at[i + LEAD - RING]],
                                  ssem[js]).wait()

          pltpu.async_copy(x_hbm.at[srcv.at[i + LEAD]], rows.at[js],
                           gsem[js])

        pltpu.make_async_copy(x_hbm.at[srcv.at[i]], rows.at[b],
                              gsem[b]).wait()
        scale(i, b)
        pltpu.async_copy(rows.at[b], acc.at[dstv.at[i]], ssem[b], add=True)
      return 0

    lax.fori_loop(0, NCHUNK // RING, outer, 0)
    for b in range(RING):
      pltpu.make_async_copy(rows.at[b], acc.at[dstv.at[NCHUNK - RING + b]],
                            ssem[b]).wait()
    plsc.subcore_barrier()

    @pl.when(sid < WB_TILES)
    def _writeback():
      for k in range(WB_ROWS // WB_CH):
        off = sid * WB_ROWS + k * WB_CH
        pltpu.sync_copy(acc.at[pl.ds(off, WB_CH)],
                        out_hbm.at[cid, pl.ds(off, WB_CH)])

  return prop


NBLK = 10
BR = N_NODES // NBLK

TPW = 25
NPP = N_NODES // TPW
NEG = -3.0e38


def _add2(p):
  C = p.shape[2]

  def body(p_ref, o_ref):
    o_ref[...] = p_ref[0] + p_ref[1]

  return pl.pallas_call(
      body,
      grid=(NBLK,),
      in_specs=[pl.BlockSpec((2, BR, C), lambda i: (0, i, 0))],
      out_specs=pl.BlockSpec((BR, C), lambda i: (i, 0)),
      out_shape=jax.ShapeDtypeStruct((N_NODES, C), jnp.float32),
  )(p)


def _pre1(x, W1, b1):
  def body(x_ref, w_ref, b_ref, a_ref, bb_ref):
    xb = x_ref[...]
    a_ref[...] = jnp.dot(xb, w_ref[0] - w_ref[2],
                         preferred_element_type=jnp.float32) + b_ref[...]
    bb_ref[...] = jnp.concatenate(
        [jnp.dot(xb, w_ref[1], preferred_element_type=jnp.float32),
         jnp.dot(xb, w_ref[2], preferred_element_type=jnp.float32)], axis=1)

  return pl.pallas_call(
      body,
      grid=(NBLK,),
      in_specs=[
          pl.BlockSpec((BR, 128), lambda i: (i, 0)),
          pl.BlockSpec((3, 128, 16), lambda i: (0, 0, 0)),
          pl.BlockSpec((1, 16), lambda i: (0, 0)),
      ],
      out_specs=[
          pl.BlockSpec((BR, 16), lambda i: (i, 0)),
          pl.BlockSpec((BR, 32), lambda i: (i, 0)),
      ],
      out_shape=[
          jax.ShapeDtypeStruct((N_NODES, 16), jnp.float32),
          jax.ShapeDtypeStruct((N_NODES, 32), jnp.float32),
      ],
  )(x, W1, b1.reshape(1, -1))


def _cheb_u(Xin, Tx1, p2, W, b, with_act=True):
  Cin, Cout = W.shape[1], W.shape[2]

  def body(x_ref, t1_ref, p2_ref, w_ref, b_ref, u_ref, st_ref):
    i = pl.program_id(0)
    xb = x_ref[...]
    t2 = 2.0 * (p2_ref[0] + p2_ref[1]) - xb
    h = (jnp.dot(xb, w_ref[0], preferred_element_type=jnp.float32) +
         jnp.dot(t1_ref[...], w_ref[1], preferred_element_type=jnp.float32) +
         jnp.dot(t2, w_ref[2], preferred_element_type=jnp.float32) +
         b_ref[...])
    if not with_act:
      u_ref[...] = h
      return
    u = jax.nn.softplus(h)
    u_ref[...] = u

    @pl.when(i == 0)
    def _init():
      st_ref[...] = jnp.zeros_like(st_ref)

    st_ref[0:1, :] += jnp.sum(u, axis=0, keepdims=True)
    st_ref[1:2, :] += jnp.sum(u * u, axis=0, keepdims=True)

  return pl.pallas_call(
      body,
      grid=(NBLK,),
      in_specs=[
          pl.BlockSpec((BR, Cin), lambda i: (i, 0)),
          pl.BlockSpec((BR, Cin), lambda i: (i, 0)),
          pl.BlockSpec((2, BR, Cin), lambda i: (0, i, 0)),
          pl.BlockSpec((3, Cin, Cout), lambda i: (0, 0, 0)),
          pl.BlockSpec((1, Cout), lambda i: (0, 0)),
      ],
      out_specs=[
          pl.BlockSpec((BR, Cout), lambda i: (i, 0)),
          pl.BlockSpec((2, Cout), lambda i: (0, 0)),
      ],
      out_shape=[
          jax.ShapeDtypeStruct((N_NODES, Cout), jnp.float32),
          jax.ShapeDtypeStruct((2, Cout), jnp.float32),
      ],
  )(Xin, Tx1, p2, W, b.reshape(1, -1))


def _l1_u(A, U1, pV):
  def body(a_ref, u1_ref, pv_ref, u_ref, st_ref):
    i = pl.program_id(0)
    u = jax.nn.softplus(a_ref[...] + u1_ref[...] +
                        2.0 * (pv_ref[0] + pv_ref[1]))
    u_ref[...] = u

    @pl.when(i == 0)
    def _init():
      st_ref[...] = jnp.zeros_like(st_ref)

    st_ref[0:1, :] += jnp.sum(u, axis=0, keepdims=True)
    st_ref[1:2, :] += jnp.sum(u * u, axis=0, keepdims=True)

  return pl.pallas_call(
      body,
      grid=(NBLK,),
      in_specs=[
          pl.BlockSpec((BR, 16), lambda i: (i, 0)),
          pl.BlockSpec((BR, 16), lambda i: (i, 0)),
          pl.BlockSpec((2, BR, 16), lambda i: (0, i, 0)),
      ],
      out_specs=[
          pl.BlockSpec((BR, 16), lambda i: (i, 0)),
          pl.BlockSpec((2, 16), lambda i: (0, 0)),
      ],
      out_shape=[
          jax.ShapeDtypeStruct((N_NODES, 16), jnp.float32),
          jax.ShapeDtypeStruct((2, 16), jnp.float32),
      ],
  )(A, U1, pV)


def _bn_apply(u, st, gamma, beta):
  C = u.shape[1]

  def body(u_ref, st_ref, g_ref, be_ref, o_ref):
    mu = st_ref[0:1, :] / N_NODES
    var = st_ref[1:2, :] / N_NODES - mu * mu
    s = g_ref[...] * lax.rsqrt(var + EPS)
    o_ref[...] = (u_ref[...] - mu) * s + be_ref[...]

  return pl.pallas_call(
      body,
      grid=(NBLK,),
      in_specs=[
          pl.BlockSpec((BR, C), lambda i: (i, 0)),
          pl.BlockSpec((2, C), lambda i: (0, 0)),
          pl.BlockSpec((1, C), lambda i: (0, 0)),
          pl.BlockSpec((1, C), lambda i: (0, 0)),
      ],
      out_specs=pl.BlockSpec((BR, C), lambda i: (i, 0)),
      out_shape=jax.ShapeDtypeStruct((N_NODES, C), jnp.float32),
  )(u, st, gamma.reshape(1, -1), beta.reshape(1, -1))


def _make_pool():
  mesh = plsc.VectorSubcoreMesh(
      core_axis_name="c", subcore_axis_name="s", num_cores=NC, num_subcores=NS)

  @functools.partial(
      pl.kernel,
      out_type=(jax.ShapeDtypeStruct((TPW, G_GRAPHS, 128), jnp.float32),
                jax.ShapeDtypeStruct((TPW, G_GRAPHS, 128), jnp.float32)),
      mesh=mesh,
      scratch_types=[
          pltpu.VMEM((NPP, 128), jnp.float32),
          pltpu.VMEM((NPP,), jnp.int32),
          pltpu.VMEM((G_GRAPHS, 128), jnp.float32),
          pltpu.VMEM((G_GRAPHS, 128), jnp.float32),
          pltpu.SemaphoreType.DMA,
      ],
      compiler_params=pltpu.CompilerParams(use_tc_tiling_on_sc=False),
  )
  def pool(h_hbm, bi_hbm, omax_hbm, osum_hbm, hbuf, bib, amax, asum, msem):
    cid = lax.axis_index("c")
    sid = lax.axis_index("s")
    tid = sid * NC + cid

    @pl.when(tid < TPW)
    def _work():
      pltpu.async_copy(h_hbm.at[pl.ds(tid * NPP, NPP)], hbuf, msem)
      pltpu.async_copy(bi_hbm.at[pl.ds(tid * NPP, NPP)], bib, msem)
      neg = jnp.full((L,), NEG, jnp.float32)
      zeros = jnp.zeros((L,), jnp.float32)

      def ibody(r, _):
        for c in range(128 // L):
          amax[r, pl.ds(c * L, L)] = neg
          asum[r, pl.ds(c * L, L)] = zeros
        return 0

      lax.fori_loop(0, G_GRAPHS, ibody, 0)
      pltpu.make_async_copy(h_hbm.at[pl.ds(tid * NPP, NPP)], hbuf, msem).wait()
      pltpu.make_async_copy(bi_hbm.at[pl.ds(tid * NPP, NPP)], bib, msem).wait()

      def gbody(g, _):
        b16 = bib[pl.ds(g * L, L)]
        for j in range(L):
          b = b16[j]
          n = g * L + j
          for c in range(128 // L):
            sl = pl.ds(c * L, L)
            v = hbuf[n, sl]
            amax[b, sl] = jnp.maximum(amax[b, sl], v)
            asum[b, sl] = asum[b, sl] + v
        return 0

      lax.fori_loop(0, NPP // L, gbody, 0)
      pltpu.sync_copy(amax, omax_hbm.at[tid])
      pltpu.sync_copy(asum, osum_hbm.at[tid])

  return pool


def _head(pmax, psum, bi, Wd, bd):
  def body(pm_ref, ps_ref, bi_ref, wd_ref, bd_ref, o_ref):
    m = jnp.max(pm_ref[...], axis=0)
    s = jnp.sum(ps_ref[...], axis=0)
    gids = lax.broadcasted_iota(jnp.int32, (G_GRAPHS, N_NODES), 0)
    cnt = jnp.sum((gids == bi_ref[...]).astype(jnp.float32), axis=1)
    cnt = jnp.maximum(cnt, 1.0)
    mean = s / cnt[:, None]
    pooled = jnp.concatenate([m, mean], axis=1)
    logits = jnp.dot(pooled, wd_ref[...],
                     preferred_element_type=jnp.float32) + bd_ref[...]
    mx = jnp.max(logits, axis=-1, keepdims=True)
    lse = mx + jnp.log(jnp.sum(jnp.exp(logits - mx), axis=-1, keepdims=True))
    o_ref[...] = logits - lse

  return pl.pallas_call(
      body,
      out_shape=jax.ShapeDtypeStruct((G_GRAPHS, 4), jnp.float32),
  )(pmax, psum, bi.reshape(1, -1), Wd, bd.reshape(1, -1))


def kernel(x, edge_index, batch_index, edge_weight, W1, b1, W2, b2, W3, b3,
           W4, b4, W5, b5, g1, be1, g2, be2, g3, be3, g4, be4, Wd, bd):
  row, col = edge_index[0], edge_index[1]
  norm_w = _make_degnorm()(row, col, edge_weight)

  src_r = col.reshape(NW, NCHUNK, K_E)
  dst_r = row.reshape(NW, NCHUNK, K_E)
  w_r = norm_w.reshape(NW, NCHUNK, K_E)

  props = {}

  def Pp(X):
    C = X.shape[1]
    if C not in props:
      props[C] = _make_prop(C)
    return props[C](X, src_r, dst_r, w_r)

  def layer(X, W, b, gamma, beta):
    p1 = Pp(X)
    Tx1 = _add2(p1)
    p2 = Pp(Tx1)
    u, st = _cheb_u(X, Tx1, p2, W, b)
    return _bn_apply(u, st, gamma, beta)

  A, B = _pre1(x, W1, b1)
  U = _add2(Pp(B))
  pV = Pp(U[:, 16:])
  u1, st1 = _l1_u(A, U[:, :16], pV)
  h = _bn_apply(u1, st1, g1, be1)

  h = layer(h, W2, b2, g2, be2)
  h = layer(h, W3, b3, g3, be3)
  h = layer(h, W4, b4, g4, be4)

  Xa, Xb = h[:, :64], h[:, 64:]
  T1a, T1b = _add2(Pp(Xa)), _add2(Pp(Xb))
  Tx1 = jnp.concatenate([T1a, T1b], axis=1)
  p2 = jnp.concatenate([Pp(T1a), Pp(T1b)], axis=2)
  h5, _ = _cheb_u(h, Tx1, p2, W5, b5, with_act=False)

  pmax, psum = _make_pool()(h5, batch_index)
  return _head(pmax, psum, batch_index, Wd, bd)


# speedup vs baseline: 1.5615x; 1.0100x over previous
import functools

import jax
import jax.numpy as jnp
from jax import lax
from jax.experimental import pallas as pl
from jax.experimental.pallas import tpu as pltpu
from jax.experimental.pallas import tpu_sc as plsc

N_NODES = 10000
N_EDGES = 320000
G_GRAPHS = 64
EPS = 1e-5

NC, NS, L = 2, 16, 16
NW = NC * NS
EPT = N_EDGES // NW
K_E = 80
NCHUNK = 125
RING = 5
LEAD = 3
WB_TILES = 10
WB_ROWS = N_NODES // WB_TILES
WB_CH = 200


N_PAD = 10240
NPT = N_PAD // NS
EPG = 2 * EPT


def _make_degnorm():
  mesh = plsc.VectorSubcoreMesh(
      core_axis_name="c", subcore_axis_name="s", num_cores=NC, num_subcores=NS)

  @functools.partial(
      pl.kernel,
      out_type=jax.ShapeDtypeStruct((N_EDGES,), jnp.float32),
      mesh=mesh,
      scratch_types=[
          pltpu.VMEM((EPG,), jnp.int32),
          pltpu.VMEM((EPG,), jnp.float32),
          pltpu.VMEM((N_PAD,), jnp.float32),
          pltpu.VMEM((NPT,), jnp.float32),
          pltpu.VMEM((N_PAD,), jnp.float32),
          pltpu.VMEM((EPT,), jnp.int32),
          pltpu.VMEM((EPT,), jnp.int32),
          pltpu.VMEM((EPT,), jnp.float32),
          pltpu.VMEM((EPT,), jnp.float32),
          pltpu.VMEM_SHARED((NS, N_PAD), jnp.float32),
          pltpu.VMEM_SHARED((N_PAD,), jnp.float32),
          pltpu.SemaphoreType.DMA,
      ],
      compiler_params=pltpu.CompilerParams(use_tc_tiling_on_sc=False,
                                           needs_layout_passes=False),
  )
  def degnorm(dst_hbm, src_hbm, w_hbm, out_hbm, dv2, ev2, dlocal, slice_v,
              dinvv, srct, dstt, wt, outv, slab, dinv_sh, msem):
    cid = lax.axis_index("c")
    sid = lax.axis_index("s")
    tid = sid * NC + cid
    zeros = jnp.zeros((L,), jnp.float32)

    pltpu.async_copy(dst_hbm.at[pl.ds(sid * EPG, EPG)], dv2, msem)
    pltpu.async_copy(w_hbm.at[pl.ds(sid * EPG, EPG)], ev2, msem)
    pltpu.async_copy(src_hbm.at[pl.ds(tid * EPT, EPT)], srct, msem)
    pltpu.async_copy(dst_hbm.at[pl.ds(tid * EPT, EPT)], dstt, msem)
    pltpu.async_copy(w_hbm.at[pl.ds(tid * EPT, EPT)], wt, msem)

    def zbody(i, _):
      dlocal[pl.ds(i * L, L)] = zeros
      return 0

    lax.fori_loop(0, N_PAD // L, zbody, 0)
    pltpu.make_async_copy(dst_hbm.at[pl.ds(sid * EPG, EPG)], dv2, msem).wait()
    pltpu.make_async_copy(w_hbm.at[pl.ds(sid * EPG, EPG)], ev2, msem).wait()

    def deg_body(i, _):
      sl = pl.ds(i * L, L)
      plsc.addupdate_scatter(dlocal, [dv2[sl]], ev2[sl])
      return 0

    lax.fori_loop(0, EPG // L, deg_body, 0)
    pltpu.sync_copy(dlocal, slab.at[sid])
    plsc.subcore_barrier()

    base = sid * NPT

    def zslice(i, _):
      slice_v[pl.ds(i * L, L)] = zeros
      return 0

    lax.fori_loop(0, NPT // L, zslice, 0)
    for k in range(NS):
      pltpu.sync_copy(slab.at[k, pl.ds(base, NPT)], dlocal.at[pl.ds(0, NPT)])

      def radd(i, _):
        sl = pl.ds(i * L, L)
        slice_v[sl] = slice_v[sl] + dlocal[sl]
        return 0

      lax.fori_loop(0, NPT // L, radd, 0)

    def rsqrt_body(i, _):
      sl = pl.ds(i * L, L)
      d = slice_v[sl]
      x = jnp.maximum(d, 1e-12)
      bits = plsc.bitcast(x, jnp.int32)
      y = plsc.bitcast(0x5F3759DF - lax.shift_right_logical(bits, 1),
                       jnp.float32)
      for _ in range(3):
        y = y * (1.5 - 0.5 * x * y * y)
      slice_v[sl] = jnp.where(d > 0, y, 0.0)
      return 0

    lax.fori_loop(0, NPT // L, rsqrt_body, 0)
    pltpu.sync_copy(slice_v, dinv_sh.at[pl.ds(base, NPT)])
    plsc.subcore_barrier()
    pltpu.sync_copy(dinv_sh, dinvv)
    pltpu.make_async_copy(src_hbm.at[pl.ds(tid * EPT, EPT)], srct, msem).wait()
    pltpu.make_async_copy(dst_hbm.at[pl.ds(tid * EPT, EPT)], dstt, msem).wait()
    pltpu.make_async_copy(w_hbm.at[pl.ds(tid * EPT, EPT)], wt, msem).wait()

    def norm_body(i, _):
      sl = pl.ds(i * L, L)
      a = plsc.load_gather(dinvv, [dstt[sl]])
      b = plsc.load_gather(dinvv, [srct[sl]])
      outv[sl] = (0.0 - a) * wt[sl] * b
      return 0

    lax.fori_loop(0, EPT // L, norm_body, 0)
    pltpu.sync_copy(outv, out_hbm.at[pl.ds(tid * EPT, EPT)])

  return degnorm


def _make_prop(C):
  mesh = plsc.VectorSubcoreMesh(
      core_axis_name="c", subcore_axis_name="s", num_cores=NC, num_subcores=NS)

  @functools.partial(
      pl.kernel,
      out_type=jax.ShapeDtypeStruct((NC, N_NODES, C), jnp.float32),
      mesh=mesh,
      scratch_types=[
          pltpu.VMEM((NCHUNK, K_E), jnp.int32),
          pltpu.VMEM((NCHUNK, K_E), jnp.int32),
          pltpu.VMEM((NCHUNK, K_E), jnp.float32),
          pltpu.VMEM((RING, K_E, C), jnp.float32),
          pltpu.VMEM((WB_CH, C), jnp.float32),
          pltpu.VMEM_SHARED((N_NODES, C), jnp.float32),
      ] + [pltpu.SemaphoreType.DMA] * (2 * RING) + [
          pltpu.SemaphoreType.DMA,
      ],
      compiler_params=pltpu.CompilerParams(use_tc_tiling_on_sc=False),
  )
  def prop(x_hbm, src_hbm, dst_hbm, w_hbm, out_hbm, srcv, dstv, wv, rows, zb,
           acc, *sems):
    gsem = sems[:RING]
    ssem = sems[RING:2 * RING]
    msem = sems[2 * RING]
    cid = lax.axis_index("c")
    sid = lax.axis_index("s")
    tid = sid * NC + cid
    zeros = jnp.zeros((L,), jnp.float32)

    pltpu.async_copy(src_hbm.at[tid], srcv, msem)
    pltpu.async_copy(dst_hbm.at[tid], dstv, msem)
    pltpu.async_copy(w_hbm.at[tid], wv, msem)

    def zrow(r, _):
      for c in range(C // L):
        zb[r, pl.ds(c * L, L)] = zeros
      return 0

    lax.fori_loop(0, WB_CH, zrow, 0)

    @pl.when(sid < WB_TILES)
    def _zero():
      for k in range(WB_ROWS // WB_CH):
        pltpu.async_copy(zb, acc.at[pl.ds(sid * WB_ROWS + k * WB_CH, WB_CH)],
                         gsem[0])
      for k in range(WB_ROWS // WB_CH):
        pltpu.make_async_copy(
            zb, acc.at[pl.ds(sid * WB_ROWS + k * WB_CH, WB_CH)],
            gsem[0]).wait()

    pltpu.make_async_copy(src_hbm.at[tid], srcv, msem).wait()
    pltpu.make_async_copy(dst_hbm.at[tid], dstv, msem).wait()
    pltpu.make_async_copy(w_hbm.at[tid], wv, msem).wait()
    plsc.subcore_barrier()

    def scale(i, s):
      def gbody(g, _):
        w16 = wv[i, pl.ds(g * L, L)]
        for j in range(L):
          ws = jnp.full((L,), w16[j], jnp.float32)
          e = g * L + j
          for c in range(C // L):
            sl = pl.ds(c * L, L)
            rows[s, e, sl] = rows[s, e, sl] * ws
        return 0

      lax.fori_loop(0, K_E // L, gbody, 0)

    for b in range(LEAD):
      pltpu.async_copy(x_hbm.at[srcv.at[b]], rows.at[b], gsem[b])

    def outer(o, _):
      for b in range(RING):
        i = RING * o + b
        js = (b + LEAD) % RING

        @pl.when(i + LEAD < NCHUNK)
        def _issue():
          @pl.when(i >= RING - LEAD)
          def _drain():
            pltpu.make_async_copy(rows.at[js],
                                  acc.at[dstv.at[i + LEAD - RING]],
                                  ssem[js]).wait()

          pltpu.async_copy(x_hbm.at[srcv.at[i + LEAD]], rows.at[js],
                           gsem[js])

        pltpu.make_async_copy(x_hbm.at[srcv.at[i]], rows.at[b],
                              gsem[b]).wait()
        scale(i, b)
        pltpu.async_copy(rows.at[b], acc.at[dstv.at[i]], ssem[b], add=True)
      return 0

    lax.fori_loop(0, NCHUNK // RING, outer, 0)
    for b in range(RING):
      pltpu.make_async_copy(rows.at[b], acc.at[dstv.at[NCHUNK - RING + b]],
                            ssem[b]).wait()
    plsc.subcore_barrier()

    @pl.when(sid < WB_TILES)
    def _writeback():
      for k in range(WB_ROWS // WB_CH):
        off = sid * WB_ROWS + k * WB_CH
        pltpu.async_copy(acc.at[pl.ds(off, WB_CH)],
                         out_hbm.at[cid, pl.ds(off, WB_CH)], gsem[0])
      for k in range(WB_ROWS // WB_CH):
        off = sid * WB_ROWS + k * WB_CH
        pltpu.make_async_copy(acc.at[pl.ds(off, WB_CH)],
                              out_hbm.at[cid, pl.ds(off, WB_CH)],
                              gsem[0]).wait()

  return prop


NBLK = 10
BR = N_NODES // NBLK

TPW = 25
NPP = N_NODES // TPW
NEG = -3.0e38


def _add2(p):
  C = p.shape[2]

  def body(p_ref, o_ref):
    o_ref[...] = p_ref[0] + p_ref[1]

  return pl.pallas_call(
      body,
      grid=(NBLK,),
      in_specs=[pl.BlockSpec((2, BR, C), lambda i: (0, i, 0))],
      out_specs=pl.BlockSpec((BR, C), lambda i: (i, 0)),
      out_shape=jax.ShapeDtypeStruct((N_NODES, C), jnp.float32),
  )(p)


def _pre1(x, W1, b1):
  def body(x_ref, w_ref, b_ref, a_ref, bb_ref):
    xb = x_ref[...]
    a_ref[...] = jnp.dot(xb, w_ref[0] - w_ref[2],
                         preferred_element_type=jnp.float32) + b_ref[...]
    bb_ref[...] = jnp.concatenate(
        [jnp.dot(xb, w_ref[1], preferred_element_type=jnp.float32),
         jnp.dot(xb, w_ref[2], preferred_element_type=jnp.float32)], axis=1)

  return pl.pallas_call(
      body,
      grid=(NBLK,),
      in_specs=[
          pl.BlockSpec((BR, 128), lambda i: (i, 0)),
          pl.BlockSpec((3, 128, 16), lambda i: (0, 0, 0)),
          pl.BlockSpec((1, 16), lambda i: (0, 0)),
      ],
      out_specs=[
          pl.BlockSpec((BR, 16), lambda i: (i, 0)),
          pl.BlockSpec((BR, 32), lambda i: (i, 0)),
      ],
      out_shape=[
          jax.ShapeDtypeStruct((N_NODES, 16), jnp.float32),
          jax.ShapeDtypeStruct((N_NODES, 32), jnp.float32),
      ],
  )(x, W1, b1.reshape(1, -1))


def _cheb_u(Xin, Tx1, p2, W, b, with_act=True):
  Cin, Cout = W.shape[1], W.shape[2]

  def body(x_ref, t1_ref, p2_ref, w_ref, b_ref, u_ref, st_ref):
    i = pl.program_id(0)
    xb = x_ref[...]
    t2 = 2.0 * (p2_ref[0] + p2_ref[1]) - xb
    h = (jnp.dot(xb, w_ref[0], preferred_element_type=jnp.float32) +
         jnp.dot(t1_ref[...], w_ref[1], preferred_element_type=jnp.float32) +
         jnp.dot(t2, w_ref[2], preferred_element_type=jnp.float32) +
         b_ref[...])
    if not with_act:
      u_ref[...] = h
      return
    u = jax.nn.softplus(h)
    u_ref[...] = u

    @pl.when(i == 0)
    def _init():
      st_ref[...] = jnp.zeros_like(st_ref)

    st_ref[0:1, :] += jnp.sum(u, axis=0, keepdims=True)
    st_ref[1:2, :] += jnp.sum(u * u, axis=0, keepdims=True)

  return pl.pallas_call(
      body,
      grid=(NBLK,),
      in_specs=[
          pl.BlockSpec((BR, Cin), lambda i: (i, 0)),
          pl.BlockSpec((BR, Cin), lambda i: (i, 0)),
          pl.BlockSpec((2, BR, Cin), lambda i: (0, i, 0)),
          pl.BlockSpec((3, Cin, Cout), lambda i: (0, 0, 0)),
          pl.BlockSpec((1, Cout), lambda i: (0, 0)),
      ],
      out_specs=[
          pl.BlockSpec((BR, Cout), lambda i: (i, 0)),
          pl.BlockSpec((2, Cout), lambda i: (0, 0)),
      ],
      out_shape=[
          jax.ShapeDtypeStruct((N_NODES, Cout), jnp.float32),
          jax.ShapeDtypeStruct((2, Cout), jnp.float32),
      ],
  )(Xin, Tx1, p2, W, b.reshape(1, -1))


def _l1_u(A, U1, pV):
  def body(a_ref, u1_ref, pv_ref, u_ref, st_ref):
    i = pl.program_id(0)
    u = jax.nn.softplus(a_ref[...] + u1_ref[...] +
                        2.0 * (pv_ref[0] + pv_ref[1]))
    u_ref[...] = u

    @pl.when(i == 0)
    def _init():
      st_ref[...] = jnp.zeros_like(st_ref)

    st_ref[0:1, :] += jnp.sum(u, axis=0, keepdims=True)
    st_ref[1:2, :] += jnp.sum(u * u, axis=0, keepdims=True)

  return pl.pallas_call(
      body,
      grid=(NBLK,),
      in_specs=[
          pl.BlockSpec((BR, 16), lambda i: (i, 0)),
          pl.BlockSpec((BR, 16), lambda i: (i, 0)),
          pl.BlockSpec((2, BR, 16), lambda i: (0, i, 0)),
      ],
      out_specs=[
          pl.BlockSpec((BR, 16), lambda i: (i, 0)),
          pl.BlockSpec((2, 16), lambda i: (0, 0)),
      ],
      out_shape=[
          jax.ShapeDtypeStruct((N_NODES, 16), jnp.float32),
          jax.ShapeDtypeStruct((2, 16), jnp.float32),
      ],
  )(A, U1, pV)


def _bn_apply(u, st, gamma, beta):
  C = u.shape[1]

  def body(u_ref, st_ref, g_ref, be_ref, o_ref):
    mu = st_ref[0:1, :] / N_NODES
    var = st_ref[1:2, :] / N_NODES - mu * mu
    s = g_ref[...] * lax.rsqrt(var + EPS)
    o_ref[...] = (u_ref[...] - mu) * s + be_ref[...]

  return pl.pallas_call(
      body,
      grid=(NBLK,),
      in_specs=[
          pl.BlockSpec((BR, C), lambda i: (i, 0)),
          pl.BlockSpec((2, C), lambda i: (0, 0)),
          pl.BlockSpec((1, C), lambda i: (0, 0)),
          pl.BlockSpec((1, C), lambda i: (0, 0)),
      ],
      out_specs=pl.BlockSpec((BR, C), lambda i: (i, 0)),
      out_shape=jax.ShapeDtypeStruct((N_NODES, C), jnp.float32),
  )(u, st, gamma.reshape(1, -1), beta.reshape(1, -1))


def _make_pool():
  mesh = plsc.VectorSubcoreMesh(
      core_axis_name="c", subcore_axis_name="s", num_cores=NC, num_subcores=NS)

  @functools.partial(
      pl.kernel,
      out_type=(jax.ShapeDtypeStruct((TPW, G_GRAPHS, 128), jnp.float32),
                jax.ShapeDtypeStruct((TPW, G_GRAPHS, 128), jnp.float32)),
      mesh=mesh,
      scratch_types=[
          pltpu.VMEM((NPP, 128), jnp.float32),
          pltpu.VMEM((NPP,), jnp.int32),
          pltpu.VMEM((G_GRAPHS, 128), jnp.float32),
          pltpu.VMEM((G_GRAPHS, 128), jnp.float32),
          pltpu.SemaphoreType.DMA,
      ],
      compiler_params=pltpu.CompilerParams(use_tc_tiling_on_sc=False),
  )
  def pool(h_hbm, bi_hbm, omax_hbm, osum_hbm, hbuf, bib, amax, asum, msem):
    cid = lax.axis_index("c")
    sid = lax.axis_index("s")
    tid = sid * NC + cid

    @pl.when(tid < TPW)
    def _work():
      pltpu.async_copy(h_hbm.at[pl.ds(tid * NPP, NPP)], hbuf, msem)
      pltpu.async_copy(bi_hbm.at[pl.ds(tid * NPP, NPP)], bib, msem)
      neg = jnp.full((L,), NEG, jnp.float32)
      zeros = jnp.zeros((L,), jnp.float32)

      def ibody(r, _):
        for c in range(128 // L):
          amax[r, pl.ds(c * L, L)] = neg
          asum[r, pl.ds(c * L, L)] = zeros
        return 0

      lax.fori_loop(0, G_GRAPHS, ibody, 0)
      pltpu.make_async_copy(h_hbm.at[pl.ds(tid * NPP, NPP)], hbuf, msem).wait()
      pltpu.make_async_copy(bi_hbm.at[pl.ds(tid * NPP, NPP)], bib, msem).wait()

      def gbody(g, _):
        b16 = bib[pl.ds(g * L, L)]
        for j in range(L):
          b = b16[j]
          n = g * L + j
          for c in range(128 // L):
            sl = pl.ds(c * L, L)
            v = hbuf[n, sl]
            amax[b, sl] = jnp.maximum(amax[b, sl], v)
            asum[b, sl] = asum[b, sl] + v
        return 0

      lax.fori_loop(0, NPP // L, gbody, 0)
      pltpu.sync_copy(amax, omax_hbm.at[tid])
      pltpu.sync_copy(asum, osum_hbm.at[tid])

  return pool


def _head(pmax, psum, bi, Wd, bd):
  def body(pm_ref, ps_ref, bi_ref, wd_ref, bd_ref, o_ref):
    m = jnp.max(pm_ref[...], axis=0)
    s = jnp.sum(ps_ref[...], axis=0)
    gids = lax.broadcasted_iota(jnp.int32, (G_GRAPHS, N_NODES), 0)
    cnt = jnp.sum((gids == bi_ref[...]).astype(jnp.float32), axis=1)
    cnt = jnp.maximum(cnt, 1.0)
    mean = s / cnt[:, None]
    pooled = jnp.concatenate([m, mean], axis=1)
    logits = jnp.dot(pooled, wd_ref[...],
                     preferred_element_type=jnp.float32) + bd_ref[...]
    mx = jnp.max(logits, axis=-1, keepdims=True)
    lse = mx + jnp.log(jnp.sum(jnp.exp(logits - mx), axis=-1, keepdims=True))
    o_ref[...] = logits - lse

  return pl.pallas_call(
      body,
      out_shape=jax.ShapeDtypeStruct((G_GRAPHS, 4), jnp.float32),
  )(pmax, psum, bi.reshape(1, -1), Wd, bd.reshape(1, -1))


def kernel(x, edge_index, batch_index, edge_weight, W1, b1, W2, b2, W3, b3,
           W4, b4, W5, b5, g1, be1, g2, be2, g3, be3, g4, be4, Wd, bd):
  row, col = edge_index[0], edge_index[1]
  norm_w = _make_degnorm()(row, col, edge_weight)

  src_r = col.reshape(NW, NCHUNK, K_E)
  dst_r = row.reshape(NW, NCHUNK, K_E)
  w_r = norm_w.reshape(NW, NCHUNK, K_E)

  props = {}

  def Pp(X):
    C = X.shape[1]
    if C not in props:
      props[C] = _make_prop(C)
    return props[C](X, src_r, dst_r, w_r)

  def layer(X, W, b, gamma, beta):
    p1 = Pp(X)
    Tx1 = _add2(p1)
    p2 = Pp(Tx1)
    u, st = _cheb_u(X, Tx1, p2, W, b)
    return _bn_apply(u, st, gamma, beta)

  A, B = _pre1(x, W1, b1)
  U = _add2(Pp(B))
  pV = Pp(U[:, 16:])
  u1, st1 = _l1_u(A, U[:, :16], pV)
  h = _bn_apply(u1, st1, g1, be1)

  h = layer(h, W2, b2, g2, be2)
  h = layer(h, W3, b3, g3, be3)
  h = layer(h, W4, b4, g4, be4)

  Xa, Xb = h[:, :64], h[:, 64:]
  T1a, T1b = _add2(Pp(Xa)), _add2(Pp(Xb))
  Tx1 = jnp.concatenate([T1a, T1b], axis=1)
  p2 = jnp.concatenate([Pp(T1a), Pp(T1b)], axis=2)
  h5, _ = _cheb_u(h, Tx1, p2, W5, b5, with_act=False)

  pmax, psum = _make_pool()(h5, batch_index)
  return _head(pmax, psum, batch_index, Wd, bd)
